# baseline (speedup 1.0000x reference)
import functools

import jax
import jax.numpy as jnp
from jax import lax
from jax.experimental import pallas as pl
from jax.experimental.pallas import tpu as pltpu
from jax.experimental.pallas import tpu_sc as plsc

N = 10000
NP = 10240
E = 320000
H = 128
TW = 144
TWB = 160
RB = 1024
BE = 2000
NC = 2
NS = 16
NW = NC * NS
EPW = E // NW
CH = 80
NCH = EPW // CH
RZ = 80
NZC = N // RZ

_f32 = jnp.float32


def _silu(v):
    return v * jax.nn.sigmoid(v)


def _init_body(x_ref, possh_ref, e1_ref, e2_ref, l1w_ref, l1b_ref,
               wa_ref, wb_ref, fpa_ref, fpb_ref,
               h0_ref, ts_ref, td_ref):
    xb = x_ref[...]
    oh1 = (xb[:, 0:1] == lax.broadcasted_iota(jnp.int32, (1, 128), 1)
           ).astype(_f32)
    oh2 = (xb[:, 1:2] == lax.broadcasted_iota(jnp.int32, (1, 8), 1)
           ).astype(_f32)
    hemb = (jnp.dot(oh1, e1_ref[...], preferred_element_type=_f32)
            + jnp.dot(oh2, e2_ref[...], preferred_element_type=_f32))
    h0 = jax.nn.relu(jnp.dot(hemb, l1w_ref[...], preferred_element_type=_f32)
                     + l1b_ref[...])
    possh = possh_ref[...]
    h0_ref[...] = h0
    z16 = jnp.zeros((h0.shape[0], 16), _f32)
    ts_ref[...] = jnp.concatenate(
        [jnp.dot(h0, wa_ref[...], preferred_element_type=_f32),
         jnp.dot(h0, fpa_ref[...], preferred_element_type=_f32) + possh,
         z16], axis=1).astype(jnp.bfloat16)
    td_ref[...] = jnp.concatenate(
        [jnp.dot(h0, wb_ref[...], preferred_element_type=_f32),
         jnp.dot(h0, fpb_ref[...], preferred_element_type=_f32) + possh,
         z16], axis=1).astype(jnp.bfloat16)


def _tc_init(xp, possh, e1p, e2p, l1w, l1b2, wa, wb, fpa, fpb):
    g = NP // RB
    return pl.pallas_call(
        _init_body,
        grid=(g,),
        in_specs=[
            pl.BlockSpec((RB, 2), lambda i: (i, 0)),
            pl.BlockSpec((RB, 16), lambda i: (i, 0)),
            pl.BlockSpec((128, 128), lambda i: (0, 0)),
            pl.BlockSpec((8, 128), lambda i: (0, 0)),
            pl.BlockSpec((128, 128), lambda i: (0, 0)),
            pl.BlockSpec((1, 128), lambda i: (0, 0)),
            pl.BlockSpec((128, 128), lambda i: (0, 0)),
            pl.BlockSpec((128, 128), lambda i: (0, 0)),
            pl.BlockSpec((128, 16), lambda i: (0, 0)),
            pl.BlockSpec((128, 16), lambda i: (0, 0)),
        ],
        out_specs=[
            pl.BlockSpec((RB, 128), lambda i: (i, 0)),
            pl.BlockSpec((RB, TWB), lambda i: (i, 0)),
            pl.BlockSpec((RB, TWB), lambda i: (i, 0)),
        ],
        out_shape=[
            jax.ShapeDtypeStruct((NP, 128), _f32),
            jax.ShapeDtypeStruct((NP, TWB), jnp.bfloat16),
            jax.ShapeDtypeStruct((NP, TWB), jnp.bfloat16),
        ],
    )(xp, possh, e1p, e2p, l1w, l1b2, wa, wb, fpa, fpb)


def _edge_body(gs_ref, gd_ref, fe2w_ref, fpk_ref, fp2_ref, ppk_ref, out_ref):
    gs = gs_ref[...].astype(_f32)
    gd = gd_ref[...].astype(_f32)
    a = gs[:, 0:128] + gd[:, 0:128]
    s16 = gs[:, 128:144] + gd[:, 128:144]
    d16 = gs[:, 128:144] - gd[:, 128:144]
    dd = d16 * ppk_ref[4:5, :]
    dist = jnp.sqrt(jnp.sum(dd * dd, axis=1, keepdims=True) + 1e-12)
    s1 = _silu(a + dist * fpk_ref[0:1, :] + fpk_ref[1:2, :])
    m = _silu(jnp.dot(s1, fe2w_ref[...], preferred_element_type=_f32)
              + fpk_ref[2:3, :])
    e = jax.nn.sigmoid(jnp.sum(m * fpk_ref[3:4, :], axis=1, keepdims=True)
                       + ppk_ref[5:6, 0:1])
    tp = _silu(s16 * ppk_ref[3:4, :] + dist * ppk_ref[0:1, :] + ppk_ref[1:2, :])
    mp = _silu(jnp.dot(tp, fp2_ref[...], preferred_element_type=_f32)
               + ppk_ref[2:3, :])
    out_ref[...] = jnp.concatenate([e * m, dist * mp], axis=1)


def _tc_edge(gs, gd, fe2w, fpk, fp2p, ppk):
    g = E // BE
    return pl.pallas_call(
        _edge_body,
        grid=(g,),
        in_specs=[
            pl.BlockSpec((BE, TWB), lambda i: (i, 0)),
            pl.BlockSpec((BE, TWB), lambda i: (i, 0)),
            pl.BlockSpec((128, 128), lambda i: (0, 0)),
            pl.BlockSpec((8, 128), lambda i: (0, 0)),
            pl.BlockSpec((16, 16), lambda i: (0, 0)),
            pl.BlockSpec((8, 16), lambda i: (0, 0)),
        ],
        out_specs=pl.BlockSpec((BE, TW), lambda i: (i, 0)),
        out_shape=jax.ShapeDtypeStruct((E, TW), _f32),
    )(gs, gd, fe2w, fpk, fp2p, ppk)


def _node1_body(h_ref, h0_ref, p16_ref, pa_ref, pb_ref,
                fh1a_ref, fh1b_ref, fh2_ref, hpk_ref,
                fhp1a_ref, fhp1b_ref, fhp2_ref, ppk_ref,
                wa_ref, wb_ref, fpa_ref, fpb_ref, shift_ref,
                h1_ref, p1_ref, ts_ref, td_ref):
    part = pa_ref[0] + pb_ref[0]
    mh = part[:, 0:128]
    mp = part[:, 128:144] * (1.0 / N)
    h = h_ref[...]
    u = _silu(jnp.dot(h, fh1a_ref[...], preferred_element_type=_f32)
              + jnp.dot(mh, fh1b_ref[...], preferred_element_type=_f32)
              + hpk_ref[0:1, :])
    hn = h + jnp.dot(u, fh2_ref[...], preferred_element_type=_f32) + hpk_ref[1:2, :]
    h1 = hn + h0_ref[...]
    p = p16_ref[...]
    tu = _silu(jnp.dot(p, fhp1a_ref[...], preferred_element_type=_f32)
               + jnp.dot(mp, fhp1b_ref[...], preferred_element_type=_f32)
               + ppk_ref[0:1, :])
    pn = p + jnp.dot(tu, fhp2_ref[...], preferred_element_type=_f32) + ppk_ref[1:2, :]
    psh = jnp.dot(pn, shift_ref[...], preferred_element_type=_f32)
    h1_ref[...] = h1
    p1_ref[...] = pn
    z16 = jnp.zeros((h1.shape[0], 16), _f32)
    ts_ref[...] = jnp.concatenate(
        [jnp.dot(h1, wa_ref[...], preferred_element_type=_f32),
         jnp.dot(h1, fpa_ref[...], preferred_element_type=_f32) + psh,
         z16], axis=1).astype(jnp.bfloat16)
    td_ref[...] = jnp.concatenate(
        [jnp.dot(h1, wb_ref[...], preferred_element_type=_f32),
         jnp.dot(h1, fpb_ref[...], preferred_element_type=_f32) + psh,
         z16], axis=1).astype(jnp.bfloat16)


def _tc_node1(h, h0, p16, parts, fh1a, fh1b, fh2w, hpk,
              fhp1a, fhp1b, fhp2p, ppk, wa, wb, fpa, fpb, shiftm):
    g = NP // RB
    full = lambda r, c: pl.BlockSpec((r, c), lambda i: (0, 0))
    return pl.pallas_call(
        _node1_body,
        grid=(g,),
        in_specs=[
            pl.BlockSpec((RB, 128), lambda i: (i, 0)),
            pl.BlockSpec((RB, 128), lambda i: (i, 0)),
            pl.BlockSpec((RB, 16), lambda i: (i, 0)),
            pl.BlockSpec((1, RB, TW), lambda i: (0, i, 0)),
            pl.BlockSpec((1, RB, TW), lambda i: (1, i, 0)),
            full(128, 128), full(128, 128), full(128, 128), full(8, 128),
            full(16, 16), full(16, 16), full(16, 16), full(8, 16),
            full(128, 128), full(128, 128), full(128, 16), full(128, 16),
            full(16, 16),
        ],
        out_specs=[
            pl.BlockSpec((RB, 128), lambda i: (i, 0)),
            pl.BlockSpec((RB, 16), lambda i: (i, 0)),
            pl.BlockSpec((RB, TWB), lambda i: (i, 0)),
            pl.BlockSpec((RB, TWB), lambda i: (i, 0)),
        ],
        out_shape=[
            jax.ShapeDtypeStruct((NP, 128), _f32),
            jax.ShapeDtypeStruct((NP, 16), _f32),
            jax.ShapeDtypeStruct((NP, TWB), jnp.bfloat16),
            jax.ShapeDtypeStruct((NP, TWB), jnp.bfloat16),
        ],
    )(h, h0, p16, parts, parts, fh1a, fh1b, fh2w, hpk,
      fhp1a, fhp1b, fhp2p, ppk, wa, wb, fpa, fpb, shiftm)


def _node2_body(h_ref, h0_ref, pa_ref, pb_ref,
                fh1a_ref, fh1b_ref, fh2_ref, hpk_ref, out_ref):
    i = pl.program_id(0)
    part = pa_ref[0] + pb_ref[0]
    mh = part[:, 0:128]
    h = h_ref[...]
    u = _silu(jnp.dot(h, fh1a_ref[...], preferred_element_type=_f32)
              + jnp.dot(mh, fh1b_ref[...], preferred_element_type=_f32)
              + hpk_ref[0:1, :])
    hn = h + jnp.dot(u, fh2_ref[...], preferred_element_type=_f32) + hpk_ref[1:2, :]
    h2 = hn + h0_ref[...]
    rows = i * RB + lax.broadcasted_iota(jnp.int32, (RB, 1), 0)
    h2m = jnp.where(rows < N, h2, 0.0)
    s = jnp.sum(h2m, axis=0, keepdims=True) * (1.0 / N)

    @pl.when(i == 0)
    def _():
        out_ref[...] = s

    @pl.when(i > 0)
    def _():
        out_ref[...] += s


def _tc_node2(h, h0, parts, fh1a, fh1b, fh2w, hpk):
    g = NP // RB
    full = lambda r, c: pl.BlockSpec((r, c), lambda i: (0, 0))
    return pl.pallas_call(
        _node2_body,
        grid=(g,),
        in_specs=[
            pl.BlockSpec((RB, 128), lambda i: (i, 0)),
            pl.BlockSpec((RB, 128), lambda i: (i, 0)),
            pl.BlockSpec((1, RB, TW), lambda i: (0, i, 0)),
            pl.BlockSpec((1, RB, TW), lambda i: (1, i, 0)),
            full(128, 128), full(128, 128), full(128, 128), full(8, 128),
        ],
        out_specs=pl.BlockSpec((1, 128), lambda i: (0, 0)),
        out_shape=jax.ShapeDtypeStruct((1, 128), _f32),
    )(h, h0, parts, parts, fh1a, fh1b, fh2w, hpk)


@functools.cache
def _make_sc_gather():
    mesh = plsc.VectorSubcoreMesh(core_axis_name="c", subcore_axis_name="s",
                                  num_cores=NC, num_subcores=NS)

    @functools.partial(
        pl.kernel,
        out_type=[jax.ShapeDtypeStruct((E, TWB), jnp.bfloat16),
                  jax.ShapeDtypeStruct((E, TWB), jnp.bfloat16)],
        mesh=mesh,
        scratch_types=[
            pltpu.VMEM((NCH, CH), jnp.int32),
            pltpu.VMEM((NCH, CH), jnp.int32),
            pltpu.VMEM((CH, TWB), jnp.bfloat16),
            pltpu.VMEM((CH, TWB), jnp.bfloat16),
            pltpu.VMEM((CH, TWB), jnp.bfloat16),
            pltpu.VMEM((CH, TWB), jnp.bfloat16),
            pltpu.SemaphoreType.DMA,
            pltpu.SemaphoreType.DMA,
            pltpu.SemaphoreType.DMA,
            pltpu.SemaphoreType.DMA,
        ],
        compiler_params=pltpu.CompilerParams(use_tc_tiling_on_sc=False),
    )
    def sc_gather(ts_hbm, td_hbm, src_hbm, dst_hbm, gs_hbm, gd_hbm,
                  idxs_v, idxd_v, bs0, bs1, bd0, bd1, ss0, ss1, sd0, sd1):
        cid = lax.axis_index("c")
        sid = lax.axis_index("s")
        wid = sid * NC + cid
        pltpu.sync_copy(src_hbm.at[wid], idxs_v)
        pltpu.sync_copy(dst_hbm.at[wid], idxd_v)
        base = wid * EPW

        def issue(j, bs, bd, ss, sd):
            pltpu.async_copy(ts_hbm.at[idxs_v.at[j]], bs, ss)
            pltpu.async_copy(td_hbm.at[idxd_v.at[j]], bd, sd)

        def drain_write(j, bs, bd, ss, sd):
            pltpu.make_async_copy(ts_hbm.at[idxs_v.at[j]], bs, ss).wait()
            pltpu.make_async_copy(td_hbm.at[idxd_v.at[j]], bd, sd).wait()
            pltpu.sync_copy(bs, gs_hbm.at[pl.ds(base + j * CH, CH)])
            pltpu.sync_copy(bd, gd_hbm.at[pl.ds(base + j * CH, CH)])

        issue(0, bs0, bd0, ss0, sd0)

        def body(k, carry):
            g0 = 2 * k
            issue(g0 + 1, bs1, bd1, ss1, sd1)
            drain_write(g0, bs0, bd0, ss0, sd0)
            issue(g0 + 2, bs0, bd0, ss0, sd0)
            drain_write(g0 + 1, bs1, bd1, ss1, sd1)
            return carry

        lax.fori_loop(0, (NCH - 1) // 2, body, 0)
        drain_write(NCH - 1, bs0, bd0, ss0, sd0)

    return sc_gather


def _sc_gather(ts, td, src2, dst2):
    return _make_sc_gather()(ts, td, src2, dst2)


@functools.cache
def _make_sc_scatter():
    mesh = plsc.VectorSubcoreMesh(core_axis_name="c", subcore_axis_name="s",
                                  num_cores=NC, num_subcores=NS)

    @functools.partial(
        pl.kernel,
        out_type=jax.ShapeDtypeStruct((NC, NP, TW), _f32),
        mesh=mesh,
        scratch_types=[
            pltpu.VMEM_SHARED((N, TW), _f32),
            pltpu.VMEM((NCH, CH), jnp.int32),
            pltpu.VMEM((CH, TW), _f32),
            pltpu.VMEM((CH, TW), _f32),
            pltpu.SemaphoreType.DMA,
            pltpu.SemaphoreType.DMA,
        ],
        compiler_params=pltpu.CompilerParams(use_tc_tiling_on_sc=False),
    )
    def sc_scatter(contrib_hbm, src_hbm, zeros_hbm, parts_hbm,
                   acc_sh, idx_v, b0, b1, s0, s1):
        cid = lax.axis_index("c")
        sid = lax.axis_index("s")
        wid = sid * NC + cid

        for k in range((NZC + NS - 1) // NS):
            c = sid + k * NS

            @pl.when(c < NZC)
            def _(c=c):
                pltpu.sync_copy(zeros_hbm.at[pl.ds(c * RZ, RZ)], b0)
                pltpu.sync_copy(b0, acc_sh.at[pl.ds(c * RZ, RZ)])

        plsc.subcore_barrier()

        pltpu.sync_copy(src_hbm.at[wid], idx_v)
        base = wid * EPW

        def issue(j, buf, sem):
            pltpu.async_copy(contrib_hbm.at[pl.ds(base + j * CH, CH)], buf, sem)

        def drain_scatter(j, buf, sem):
            pltpu.make_async_copy(
                contrib_hbm.at[pl.ds(base + j * CH, CH)], buf, sem).wait()
            pltpu.sync_copy(buf, acc_sh.at[idx_v.at[j]], add=True)

        issue(0, b0, s0)

        def body(k, carry):
            g0 = 2 * k
            issue(g0 + 1, b1, s1)
            drain_scatter(g0, b0, s0)
            issue(g0 + 2, b0, s0)
            drain_scatter(g0 + 1, b1, s1)
            return carry

        lax.fori_loop(0, (NCH - 1) // 2, body, 0)
        drain_scatter(NCH - 1, b0, s0)
        plsc.subcore_barrier()

        for k in range((NZC + NS - 1) // NS):
            c = sid + k * NS

            @pl.when(c < NZC)
            def _(c=c):
                pltpu.sync_copy(acc_sh.at[pl.ds(c * RZ, RZ)], b0)
                pltpu.sync_copy(b0, parts_hbm.at[cid, pl.ds(c * RZ, RZ)])

    return sc_scatter


def _sc_scatter(contrib, src2, zeros144):
    return _make_sc_scatter()(contrib, src2, zeros144)


def _conv_consts(p):
    fpk = jnp.zeros((8, 128), _f32)
    fpk = fpk.at[0].set(p["fe1_w"][256])
    fpk = fpk.at[1].set(p["fe1_b"])
    fpk = fpk.at[2].set(p["fe2_b"])
    fpk = fpk.at[3].set(p["inf_w"][:, 0])
    ppk = jnp.zeros((8, 16), _f32)
    ppk = ppk.at[0, 0:3].set(p["fp1_w"][256])
    ppk = ppk.at[1, 0:3].set(p["fp1_b"])
    ppk = ppk.at[2, 0:3].set(p["fp2_b"])
    ppk = ppk.at[3, 0:3].set(1.0)
    ppk = ppk.at[4, 3:6].set(1.0)
    ppk = ppk.at[5, 0].set(p["inf_b"][0])
    fp2p = jnp.zeros((16, 16), _f32).at[0:3, 0:3].set(p["fp2_w"])
    wa = p["fe1_w"][0:128]
    wb = p["fe1_w"][128:256]
    fpa = jnp.pad(p["fp1_w"][0:128], ((0, 0), (0, 13)))
    fpb = jnp.pad(p["fp1_w"][128:256], ((0, 0), (0, 13)))
    hpk = jnp.zeros((8, 128), _f32)
    hpk = hpk.at[0].set(p["fh1_b"])
    hpk = hpk.at[1].set(p["fh2_b"])
    fh1a = p["fh1_w"][0:128]
    fh1b = p["fh1_w"][128:256]
    fhp1a = jnp.zeros((16, 16), _f32).at[0:3, 0:3].set(p["fhp1_w"][0:3])
    fhp1b = jnp.zeros((16, 16), _f32).at[0:3, 0:3].set(p["fhp1_w"][3:6])
    fhp2p = jnp.zeros((16, 16), _f32).at[0:3, 0:3].set(p["fhp2_w"])
    ppku = jnp.zeros((8, 16), _f32)
    ppku = ppku.at[0, 0:3].set(p["fhp1_b"])
    ppku = ppku.at[1, 0:3].set(p["fhp2_b"])
    return dict(fpk=fpk, ppk=ppk, fp2p=fp2p, wa=wa, wb=wb, fpa=fpa, fpb=fpb,
                hpk=hpk, fh1a=fh1a, fh1b=fh1b, fhp1a=fhp1a, fhp1b=fhp1b,
                fhp2p=fhp2p, ppku=ppku, fe2w=p["fe2_w"])


def kernel(x, edge_index, pos, emb1, emb2, l1_w, l1_b, params):
    xp = jnp.pad(x.astype(jnp.int32), ((0, NP - N), (0, 0)))
    pos = pos.astype(_f32)
    possh = jnp.pad(pos, ((0, NP - N), (3, 10)))
    p16 = jnp.pad(pos, ((0, NP - N), (0, 13)))
    src2 = edge_index[:, 0].astype(jnp.int32).reshape(NW, NCH, CH)
    dst2 = edge_index[:, 1].astype(jnp.int32).reshape(NW, NCH, CH)
    zeros144 = jnp.zeros((N, TW), _f32)
    e1p = jnp.pad(emb1, ((0, 128 - emb1.shape[0]), (0, 0)))
    e2p = jnp.pad(emb2, ((0, 8 - emb2.shape[0]), (0, 0)))
    l1b2 = l1_b[None, :]
    shiftm = jnp.zeros((16, 16), _f32).at[0, 3].set(1.0).at[1, 4].set(1.0).at[2, 5].set(1.0)

    c1 = _conv_consts(params["c1"])
    c2 = _conv_consts(params["c2"])

    h0, t1s, t1d = _tc_init(xp, possh, e1p, e2p, l1_w, l1b2,
                            c1["wa"], c1["wb"], c1["fpa"], c1["fpb"])
    gs1, gd1 = _sc_gather(t1s, t1d, src2, dst2)
    contrib1 = _tc_edge(gs1, gd1, c1["fe2w"], c1["fpk"], c1["fp2p"], c1["ppk"])
    parts1 = _sc_scatter(contrib1, src2, zeros144)
    h1, p16_1, t2s, t2d = _tc_node1(
        h0, h0, p16, parts1,
        c1["fh1a"], c1["fh1b"], params["c1"]["fh2_w"], c1["hpk"],
        c1["fhp1a"], c1["fhp1b"], c1["fhp2p"], c1["ppku"],
        c2["wa"], c2["wb"], c2["fpa"], c2["fpb"], shiftm)
    gs2, gd2 = _sc_gather(t2s, t2d, src2, dst2)
    contrib2 = _tc_edge(gs2, gd2, c2["fe2w"], c2["fpk"], c2["fp2p"], c2["ppk"])
    parts2 = _sc_scatter(contrib2, src2, zeros144)
    out = _tc_node2(h1, h0, parts2,
                    c2["fh1a"], c2["fh1b"], params["c2"]["fh2_w"], c2["hpk"])
    return out


# speedup vs baseline: 2.5226x; 2.5226x over previous
import functools

import jax
import jax.numpy as jnp
from jax import lax
from jax.experimental import pallas as pl
from jax.experimental.pallas import tpu as pltpu
from jax.experimental.pallas import tpu_sc as plsc

N = 10000
NP = 10240
E = 320000
H = 128
RB = 1024
BE = 2000
NC = 2
NS = 16
NW = NC * NS
EPW = E // NW
CH = 80
NCH = EPW // CH
RZ = 80
NZC = N // RZ

_f32 = jnp.float32


def _silu(v):
    return v * jax.nn.sigmoid(v)


def _init_body(x_ref, possh_ref, e1_ref, e2_ref, l1w_ref, l1b_ref,
               wa_ref, wb_ref, fpa_ref, fpb_ref,
               h0_ref, tsp_ref, tdp_ref, tse_ref, tde_ref):
    xb = x_ref[...]
    oh1 = (xb[:, 0:1] == lax.broadcasted_iota(jnp.int32, (1, 128), 1)
           ).astype(_f32)
    oh2 = (xb[:, 1:2] == lax.broadcasted_iota(jnp.int32, (1, 8), 1)
           ).astype(_f32)
    hemb = (jnp.dot(oh1, e1_ref[...], preferred_element_type=_f32)
            + jnp.dot(oh2, e2_ref[...], preferred_element_type=_f32))
    h0 = jax.nn.relu(jnp.dot(hemb, l1w_ref[...], preferred_element_type=_f32)
                     + l1b_ref[...])
    possh = possh_ref[...]
    h0_ref[...] = h0
    tsp_ref[...] = jnp.dot(h0, wa_ref[...], preferred_element_type=_f32)
    tdp_ref[...] = jnp.dot(h0, wb_ref[...], preferred_element_type=_f32)
    tse_ref[...] = jnp.dot(h0, fpa_ref[...], preferred_element_type=_f32) + possh
    tde_ref[...] = jnp.dot(h0, fpb_ref[...], preferred_element_type=_f32) + possh


def _tc_init(xp, possh, e1p, e2p, l1w, l1b2, wa, wb, fpa, fpb):
    g = NP // RB
    return pl.pallas_call(
        _init_body,
        grid=(g,),
        in_specs=[
            pl.BlockSpec((RB, 2), lambda i: (i, 0)),
            pl.BlockSpec((RB, 16), lambda i: (i, 0)),
            pl.BlockSpec((128, 128), lambda i: (0, 0)),
            pl.BlockSpec((8, 128), lambda i: (0, 0)),
            pl.BlockSpec((128, 128), lambda i: (0, 0)),
            pl.BlockSpec((1, 128), lambda i: (0, 0)),
            pl.BlockSpec((128, 128), lambda i: (0, 0)),
            pl.BlockSpec((128, 128), lambda i: (0, 0)),
            pl.BlockSpec((128, 16), lambda i: (0, 0)),
            pl.BlockSpec((128, 16), lambda i: (0, 0)),
        ],
        out_specs=[
            pl.BlockSpec((RB, 128), lambda i: (i, 0)),
            pl.BlockSpec((RB, 128), lambda i: (i, 0)),
            pl.BlockSpec((RB, 128), lambda i: (i, 0)),
            pl.BlockSpec((RB, 16), lambda i: (i, 0)),
            pl.BlockSpec((RB, 16), lambda i: (i, 0)),
        ],
        out_shape=[
            jax.ShapeDtypeStruct((NP, 128), _f32),
            jax.ShapeDtypeStruct((NP, 128), _f32),
            jax.ShapeDtypeStruct((NP, 128), _f32),
            jax.ShapeDtypeStruct((NP, 16), _f32),
            jax.ShapeDtypeStruct((NP, 16), _f32),
        ],
    )(xp, possh, e1p, e2p, l1w, l1b2, wa, wb, fpa, fpb)


def _edge_body(gsp_ref, gdp_ref, exc_ref,
               fe2w_ref, fpk_ref, fp2_ref, ppk_ref, outh_ref, outp_ref):
    a = gsp_ref[...] + gdp_ref[...]
    ex = exc_ref[...]
    es = ex[:, 0:16]
    ed = ex[:, 16:32]
    s16 = es + ed
    d16 = es - ed
    dd = d16 * ppk_ref[4:5, :]
    dist = jnp.sqrt(jnp.sum(dd * dd, axis=1, keepdims=True) + 1e-12)
    s1 = _silu(a + dist * fpk_ref[0:1, :] + fpk_ref[1:2, :])
    m = _silu(jnp.dot(s1, fe2w_ref[...], preferred_element_type=_f32)
              + fpk_ref[2:3, :])
    e = jax.nn.sigmoid(jnp.sum(m * fpk_ref[3:4, :], axis=1, keepdims=True)
                       + ppk_ref[5:6, 0:1])
    tp = _silu(s16 * ppk_ref[3:4, :] + dist * ppk_ref[0:1, :] + ppk_ref[1:2, :])
    mp = _silu(jnp.dot(tp, fp2_ref[...], preferred_element_type=_f32)
               + ppk_ref[2:3, :])
    outh_ref[...] = e * m
    outp_ref[...] = jnp.concatenate(
        [dist * mp, jnp.zeros((mp.shape[0], 112), _f32)], axis=1)


def _tc_edge(gsp, gdp, exc, fe2w, fpk, fp2p, ppk):
    g = E // BE
    return pl.pallas_call(
        _edge_body,
        grid=(g,),
        in_specs=[
            pl.BlockSpec((BE, 128), lambda i: (i, 0)),
            pl.BlockSpec((BE, 128), lambda i: (i, 0)),
            pl.BlockSpec((BE, 128), lambda i: (i, 0)),
            pl.BlockSpec((128, 128), lambda i: (0, 0)),
            pl.BlockSpec((8, 128), lambda i: (0, 0)),
            pl.BlockSpec((16, 16), lambda i: (0, 0)),
            pl.BlockSpec((8, 16), lambda i: (0, 0)),
        ],
        out_specs=[
            pl.BlockSpec((BE, 128), lambda i: (i, 0)),
            pl.BlockSpec((BE, 128), lambda i: (i, 0)),
        ],
        out_shape=[
            jax.ShapeDtypeStruct((E, 128), _f32),
            jax.ShapeDtypeStruct((E, 128), _f32),
        ],
    )(gsp, gdp, exc, fe2w, fpk, fp2p, ppk)


def _node1_body(h_ref, h0_ref, p16_ref, pha_ref, phb_ref, ppa_ref, ppb_ref,
                fh1a_ref, fh1b_ref, fh2_ref, hpk_ref,
                fhp1a_ref, fhp1b_ref, fhp2_ref, ppk_ref,
                wa_ref, wb_ref, fpa_ref, fpb_ref, shift_ref,
                h1_ref, p1_ref, tsp_ref, tdp_ref, tse_ref, tde_ref):
    mh = pha_ref[0] + phb_ref[0]
    mp = (ppa_ref[0][:, 0:16] + ppb_ref[0][:, 0:16]) * (1.0 / N)
    h = h_ref[...]
    u = _silu(jnp.dot(h, fh1a_ref[...], preferred_element_type=_f32)
              + jnp.dot(mh, fh1b_ref[...], preferred_element_type=_f32)
              + hpk_ref[0:1, :])
    hn = h + jnp.dot(u, fh2_ref[...], preferred_element_type=_f32) + hpk_ref[1:2, :]
    h1 = hn + h0_ref[...]
    p = p16_ref[...]
    tu = _silu(jnp.dot(p, fhp1a_ref[...], preferred_element_type=_f32)
               + jnp.dot(mp, fhp1b_ref[...], preferred_element_type=_f32)
               + ppk_ref[0:1, :])
    pn = p + jnp.dot(tu, fhp2_ref[...], preferred_element_type=_f32) + ppk_ref[1:2, :]
    psh = jnp.dot(pn, shift_ref[...], preferred_element_type=_f32)
    h1_ref[...] = h1
    p1_ref[...] = pn
    tsp_ref[...] = jnp.dot(h1, wa_ref[...], preferred_element_type=_f32)
    tdp_ref[...] = jnp.dot(h1, wb_ref[...], preferred_element_type=_f32)
    tse_ref[...] = jnp.dot(h1, fpa_ref[...], preferred_element_type=_f32) + psh
    tde_ref[...] = jnp.dot(h1, fpb_ref[...], preferred_element_type=_f32) + psh


def _tc_node1(h, h0, p16, parts_h, parts_p, fh1a, fh1b, fh2w, hpk,
              fhp1a, fhp1b, fhp2p, ppk, wa, wb, fpa, fpb, shiftm):
    g = NP // RB
    full = lambda r, c: pl.BlockSpec((r, c), lambda i: (0, 0))
    return pl.pallas_call(
        _node1_body,
        grid=(g,),
        in_specs=[
            pl.BlockSpec((RB, 128), lambda i: (i, 0)),
            pl.BlockSpec((RB, 128), lambda i: (i, 0)),
            pl.BlockSpec((RB, 16), lambda i: (i, 0)),
            pl.BlockSpec((1, RB, 128), lambda i: (0, i, 0)),
            pl.BlockSpec((1, RB, 128), lambda i: (1, i, 0)),
            pl.BlockSpec((1, RB, 128), lambda i: (0, i, 0)),
            pl.BlockSpec((1, RB, 128), lambda i: (1, i, 0)),
            full(128, 128), full(128, 128), full(128, 128), full(8, 128),
            full(16, 16), full(16, 16), full(16, 16), full(8, 16),
            full(128, 128), full(128, 128), full(128, 16), full(128, 16),
            full(16, 16),
        ],
        out_specs=[
            pl.BlockSpec((RB, 128), lambda i: (i, 0)),
            pl.BlockSpec((RB, 16), lambda i: (i, 0)),
            pl.BlockSpec((RB, 128), lambda i: (i, 0)),
            pl.BlockSpec((RB, 128), lambda i: (i, 0)),
            pl.BlockSpec((RB, 16), lambda i: (i, 0)),
            pl.BlockSpec((RB, 16), lambda i: (i, 0)),
        ],
        out_shape=[
            jax.ShapeDtypeStruct((NP, 128), _f32),
            jax.ShapeDtypeStruct((NP, 16), _f32),
            jax.ShapeDtypeStruct((NP, 128), _f32),
            jax.ShapeDtypeStruct((NP, 128), _f32),
            jax.ShapeDtypeStruct((NP, 16), _f32),
            jax.ShapeDtypeStruct((NP, 16), _f32),
        ],
    )(h, h0, p16, parts_h, parts_h, parts_p, parts_p, fh1a, fh1b, fh2w, hpk,
      fhp1a, fhp1b, fhp2p, ppk, wa, wb, fpa, fpb, shiftm)


def _node2_body(h_ref, h0_ref, pha_ref, phb_ref,
                fh1a_ref, fh1b_ref, fh2_ref, hpk_ref, out_ref):
    i = pl.program_id(0)
    mh = pha_ref[0] + phb_ref[0]
    h = h_ref[...]
    u = _silu(jnp.dot(h, fh1a_ref[...], preferred_element_type=_f32)
              + jnp.dot(mh, fh1b_ref[...], preferred_element_type=_f32)
              + hpk_ref[0:1, :])
    hn = h + jnp.dot(u, fh2_ref[...], preferred_element_type=_f32) + hpk_ref[1:2, :]
    h2 = hn + h0_ref[...]
    rows = i * RB + lax.broadcasted_iota(jnp.int32, (RB, 1), 0)
    h2m = jnp.where(rows < N, h2, 0.0)
    s = jnp.sum(h2m, axis=0, keepdims=True) * (1.0 / N)

    @pl.when(i == 0)
    def _():
        out_ref[...] = s

    @pl.when(i > 0)
    def _():
        out_ref[...] += s


def _tc_node2(h, h0, parts_h, fh1a, fh1b, fh2w, hpk):
    g = NP // RB
    full = lambda r, c: pl.BlockSpec((r, c), lambda i: (0, 0))
    return pl.pallas_call(
        _node2_body,
        grid=(g,),
        in_specs=[
            pl.BlockSpec((RB, 128), lambda i: (i, 0)),
            pl.BlockSpec((RB, 128), lambda i: (i, 0)),
            pl.BlockSpec((1, RB, 128), lambda i: (0, i, 0)),
            pl.BlockSpec((1, RB, 128), lambda i: (1, i, 0)),
            full(128, 128), full(128, 128), full(128, 128), full(8, 128),
        ],
        out_specs=pl.BlockSpec((1, 128), lambda i: (0, 0)),
        out_shape=jax.ShapeDtypeStruct((1, 128), _f32),
    )(h, h0, parts_h, parts_h, fh1a, fh1b, fh2w, hpk)


@functools.cache
def _make_sc_gather():
    mesh = plsc.VectorSubcoreMesh(core_axis_name="c", subcore_axis_name="s",
                                  num_cores=NC, num_subcores=NS)

    @functools.partial(
        pl.kernel,
        out_type=[jax.ShapeDtypeStruct((E, 128), _f32),
                  jax.ShapeDtypeStruct((E, 128), _f32),
                  jax.ShapeDtypeStruct((E, 128), _f32)],
        mesh=mesh,
        scratch_types=[
            pltpu.VMEM((NCH, CH), jnp.int32),
            pltpu.VMEM((NCH, CH), jnp.int32),
            pltpu.VMEM((CH, 128), _f32),
            pltpu.VMEM((CH, 128), _f32),
            pltpu.VMEM((CH, 16), _f32),
            pltpu.VMEM((CH, 16), _f32),
            pltpu.VMEM((CH, 128), _f32),
            pltpu.VMEM((CH, 128), _f32),
            pltpu.VMEM((CH, 16), _f32),
            pltpu.VMEM((CH, 16), _f32),
            pltpu.SemaphoreType.DMA,
            pltpu.SemaphoreType.DMA,
        ],
        compiler_params=pltpu.CompilerParams(use_tc_tiling_on_sc=False),
    )
    def sc_gather(tsp_hbm, tdp_hbm, tse_hbm, tde_hbm, src_hbm, dst_hbm,
                  gsp_hbm, gdp_hbm, exc_hbm,
                  idxs_v, idxd_v,
                  b0sp, b0dp, b0se, b0de, b1sp, b1dp, b1se, b1de,
                  sem0, sem1):
        cid = lax.axis_index("c")
        sid = lax.axis_index("s")
        wid = sid * NC + cid
        pltpu.sync_copy(src_hbm.at[wid], idxs_v)
        pltpu.sync_copy(dst_hbm.at[wid], idxd_v)
        base = wid * EPW
        tables = (tsp_hbm, tdp_hbm, tse_hbm, tde_hbm)
        idxes = (idxs_v, idxd_v, idxs_v, idxd_v)
        bufs = ((b0sp, b0dp, b0se, b0de), (b1sp, b1dp, b1se, b1de))

        def issue(j, p, sem):
            for t, b, idx in zip(tables, bufs[p], idxes):
                pltpu.async_copy(t.at[idx.at[j]], b, sem)

        def drain_write(j, p, sem):
            for t, b, idx in zip(tables, bufs[p], idxes):
                pltpu.make_async_copy(t.at[idx.at[j]], b, sem).wait()
            r = pl.ds(base + j * CH, CH)
            pltpu.sync_copy(bufs[p][0], gsp_hbm.at[r])
            pltpu.sync_copy(bufs[p][1], gdp_hbm.at[r])
            pltpu.sync_copy(bufs[p][2], exc_hbm.at[r, pl.ds(0, 16)])
            pltpu.sync_copy(bufs[p][3], exc_hbm.at[r, pl.ds(16, 16)])

        issue(0, 0, sem0)

        def body(k, carry):
            g0 = 2 * k
            issue(g0 + 1, 1, sem1)
            drain_write(g0, 0, sem0)
            issue(g0 + 2, 0, sem0)
            drain_write(g0 + 1, 1, sem1)
            return carry

        lax.fori_loop(0, (NCH - 1) // 2, body, 0)
        drain_write(NCH - 1, 0, sem0)

    return sc_gather


def _sc_gather(tsp, tdp, tse, tde, src2, dst2):
    return _make_sc_gather()(tsp, tdp, tse, tde, src2, dst2)


@functools.cache
def _make_sc_scatter():
    mesh = plsc.VectorSubcoreMesh(core_axis_name="c", subcore_axis_name="s",
                                  num_cores=NC, num_subcores=NS)

    @functools.partial(
        pl.kernel,
        out_type=[jax.ShapeDtypeStruct((NC, NP, 128), _f32),
                  jax.ShapeDtypeStruct((NC, NP, 128), _f32)],
        mesh=mesh,
        scratch_types=[
            pltpu.VMEM_SHARED((N, 128), _f32),
            pltpu.VMEM_SHARED((N, 16), _f32),
            pltpu.VMEM((NCH, CH), jnp.int32),
            pltpu.VMEM((CH, 128), _f32),
            pltpu.VMEM((CH, 128), _f32),
            pltpu.VMEM((CH, 16), _f32),
            pltpu.VMEM((CH, 16), _f32),
            pltpu.SemaphoreType.DMA,
            pltpu.SemaphoreType.DMA,
        ],
        compiler_params=pltpu.CompilerParams(use_tc_tiling_on_sc=False),
    )
    def sc_scatter(conh_hbm, conp_hbm, src_hbm, zeros_hbm, parth_hbm, partp_hbm,
                   acch_sh, accp_sh, idx_v, bh0, bh1, bp0, bp1, s0, s1):
        cid = lax.axis_index("c")
        sid = lax.axis_index("s")
        wid = sid * NC + cid

        for k in range((NZC + NS - 1) // NS):
            c = sid + k * NS

            @pl.when(c < NZC)
            def _(c=c):
                r = pl.ds(c * RZ, RZ)
                pltpu.sync_copy(zeros_hbm.at[r], bh0)
                pltpu.sync_copy(bh0, acch_sh.at[r])
                pltpu.sync_copy(zeros_hbm.at[r, pl.ds(0, 16)], bp0)
                pltpu.sync_copy(bp0, accp_sh.at[r])

        plsc.subcore_barrier()

        pltpu.sync_copy(src_hbm.at[wid], idx_v)
        base = wid * EPW

        def issue(j, bh, bp, sem):
            r = pl.ds(base + j * CH, CH)
            pltpu.async_copy(conh_hbm.at[r], bh, sem)
            pltpu.async_copy(conp_hbm.at[r, pl.ds(0, 16)], bp, sem)

        def drain_scatter(j, bh, bp, sem):
            r = pl.ds(base + j * CH, CH)
            pltpu.make_async_copy(conh_hbm.at[r], bh, sem).wait()
            pltpu.make_async_copy(conp_hbm.at[r, pl.ds(0, 16)], bp, sem).wait()
            pltpu.sync_copy(bh, acch_sh.at[idx_v.at[j]], add=True)
            pltpu.sync_copy(bp, accp_sh.at[idx_v.at[j]], add=True)

        issue(0, bh0, bp0, s0)

        def body(k, carry):
            g0 = 2 * k
            issue(g0 + 1, bh1, bp1, s1)
            drain_scatter(g0, bh0, bp0, s0)
            issue(g0 + 2, bh0, bp0, s0)
            drain_scatter(g0 + 1, bh1, bp1, s1)
            return carry

        lax.fori_loop(0, (NCH - 1) // 2, body, 0)
        drain_scatter(NCH - 1, bh0, bp0, s0)
        plsc.subcore_barrier()

        for k in range((NZC + NS - 1) // NS):
            c = sid + k * NS

            @pl.when(c < NZC)
            def _(c=c):
                r = pl.ds(c * RZ, RZ)
                pltpu.sync_copy(acch_sh.at[r], bh0)
                pltpu.sync_copy(bh0, parth_hbm.at[cid, r])
                pltpu.sync_copy(accp_sh.at[r], bp0)
                pltpu.sync_copy(bp0, partp_hbm.at[cid, r, pl.ds(0, 16)])

    return sc_scatter


def _sc_scatter(conh, conp, src2, zeros128):
    return _make_sc_scatter()(conh, conp, src2, zeros128)


def _conv_consts(p):
    fpk = jnp.zeros((8, 128), _f32)
    fpk = fpk.at[0].set(p["fe1_w"][256])
    fpk = fpk.at[1].set(p["fe1_b"])
    fpk = fpk.at[2].set(p["fe2_b"])
    fpk = fpk.at[3].set(p["inf_w"][:, 0])
    ppk = jnp.zeros((8, 16), _f32)
    ppk = ppk.at[0, 0:3].set(p["fp1_w"][256])
    ppk = ppk.at[1, 0:3].set(p["fp1_b"])
    ppk = ppk.at[2, 0:3].set(p["fp2_b"])
    ppk = ppk.at[3, 0:3].set(1.0)
    ppk = ppk.at[4, 3:6].set(1.0)
    ppk = ppk.at[5, 0].set(p["inf_b"][0])
    fp2p = jnp.zeros((16, 16), _f32).at[0:3, 0:3].set(p["fp2_w"])
    wa = p["fe1_w"][0:128]
    wb = p["fe1_w"][128:256]
    fpa = jnp.pad(p["fp1_w"][0:128], ((0, 0), (0, 13)))
    fpb = jnp.pad(p["fp1_w"][128:256], ((0, 0), (0, 13)))
    hpk = jnp.zeros((8, 128), _f32)
    hpk = hpk.at[0].set(p["fh1_b"])
    hpk = hpk.at[1].set(p["fh2_b"])
    fh1a = p["fh1_w"][0:128]
    fh1b = p["fh1_w"][128:256]
    fhp1a = jnp.zeros((16, 16), _f32).at[0:3, 0:3].set(p["fhp1_w"][0:3])
    fhp1b = jnp.zeros((16, 16), _f32).at[0:3, 0:3].set(p["fhp1_w"][3:6])
    fhp2p = jnp.zeros((16, 16), _f32).at[0:3, 0:3].set(p["fhp2_w"])
    ppku = jnp.zeros((8, 16), _f32)
    ppku = ppku.at[0, 0:3].set(p["fhp1_b"])
    ppku = ppku.at[1, 0:3].set(p["fhp2_b"])
    return dict(fpk=fpk, ppk=ppk, fp2p=fp2p, wa=wa, wb=wb, fpa=fpa, fpb=fpb,
                hpk=hpk, fh1a=fh1a, fh1b=fh1b, fhp1a=fhp1a, fhp1b=fhp1b,
                fhp2p=fhp2p, ppku=ppku, fe2w=p["fe2_w"])


def kernel(x, edge_index, pos, emb1, emb2, l1_w, l1_b, params):
    xp = jnp.pad(x.astype(jnp.int32), ((0, NP - N), (0, 0)))
    pos = pos.astype(_f32)
    possh = jnp.pad(pos, ((0, NP - N), (3, 10)))
    p16 = jnp.pad(pos, ((0, NP - N), (0, 13)))
    src2 = edge_index[:, 0].astype(jnp.int32).reshape(NW, NCH, CH)
    dst2 = edge_index[:, 1].astype(jnp.int32).reshape(NW, NCH, CH)
    zeros128 = jnp.zeros((N, 128), _f32)
    e1p = jnp.pad(emb1, ((0, 128 - emb1.shape[0]), (0, 0)))
    e2p = jnp.pad(emb2, ((0, 8 - emb2.shape[0]), (0, 0)))
    l1b2 = l1_b[None, :]
    shiftm = jnp.zeros((16, 16), _f32).at[0, 3].set(1.0).at[1, 4].set(1.0).at[2, 5].set(1.0)

    c1 = _conv_consts(params["c1"])
    c2 = _conv_consts(params["c2"])

    h0, t1sp, t1dp, t1se, t1de = _tc_init(xp, possh, e1p, e2p, l1_w, l1b2,
                                          c1["wa"], c1["wb"], c1["fpa"], c1["fpb"])
    gsp1, gdp1, exc1 = _sc_gather(t1sp, t1dp, t1se, t1de, src2, dst2)
    ch1, cp1 = _tc_edge(gsp1, gdp1, exc1,
                        c1["fe2w"], c1["fpk"], c1["fp2p"], c1["ppk"])
    ph1, pp1 = _sc_scatter(ch1, cp1, src2, zeros128)
    h1, p16_1, t2sp, t2dp, t2se, t2de = _tc_node1(
        h0, h0, p16, ph1, pp1,
        c1["fh1a"], c1["fh1b"], params["c1"]["fh2_w"], c1["hpk"],
        c1["fhp1a"], c1["fhp1b"], c1["fhp2p"], c1["ppku"],
        c2["wa"], c2["wb"], c2["fpa"], c2["fpb"], shiftm)
    gsp2, gdp2, exc2 = _sc_gather(t2sp, t2dp, t2se, t2de, src2, dst2)
    ch2, cp2 = _tc_edge(gsp2, gdp2, exc2,
                        c2["fe2w"], c2["fpk"], c2["fp2p"], c2["ppk"])
    ph2, _pp2 = _sc_scatter(ch2, cp2, src2, zeros128)
    out = _tc_node2(h1, h0, ph2,
                    c2["fh1a"], c2["fh1b"], params["c2"]["fh2_w"], c2["hpk"])
    return out


# speedup vs baseline: 2.6675x; 1.0574x over previous
import functools

import jax
import jax.numpy as jnp
from jax import lax
from jax.experimental import pallas as pl
from jax.experimental.pallas import tpu as pltpu
from jax.experimental.pallas import tpu_sc as plsc

N = 10000
NP = 10240
E = 320000
H = 128
RB = 1024
BE = 2000
NC = 2
NS = 16
NW = NC * NS
EPW = E // NW
CH = 80
NCH = EPW // CH
K = 5
SLAB = E // K
EPWS = SLAB // NW
NCHS = EPWS // CH
RZ = 80
NZC = N // RZ

_f32 = jnp.float32


def _silu(v):
    return v * jax.nn.sigmoid(v)


def _init_body(x_ref, possh_ref, e1_ref, e2_ref, l1w_ref, l1b_ref,
               wa_ref, wb_ref, fpa_ref, fpb_ref,
               h0_ref, tsp_ref, tdp_ref, tse_ref, tde_ref):
    xb = x_ref[...]
    oh1 = (xb[:, 0:1] == lax.broadcasted_iota(jnp.int32, (1, 128), 1)
           ).astype(_f32)
    oh2 = (xb[:, 1:2] == lax.broadcasted_iota(jnp.int32, (1, 8), 1)
           ).astype(_f32)
    hemb = (jnp.dot(oh1, e1_ref[...], preferred_element_type=_f32)
            + jnp.dot(oh2, e2_ref[...], preferred_element_type=_f32))
    h0 = jax.nn.relu(jnp.dot(hemb, l1w_ref[...], preferred_element_type=_f32)
                     + l1b_ref[...])
    possh = possh_ref[...]
    h0_ref[...] = h0
    tsp_ref[...] = jnp.dot(h0, wa_ref[...], preferred_element_type=_f32)
    tdp_ref[...] = jnp.dot(h0, wb_ref[...], preferred_element_type=_f32)
    tse_ref[...] = jnp.dot(h0, fpa_ref[...], preferred_element_type=_f32) + possh
    tde_ref[...] = jnp.dot(h0, fpb_ref[...], preferred_element_type=_f32) + possh


def _tc_init(xp, possh, e1p, e2p, l1w, l1b2, wa, wb, fpa, fpb):
    g = NP // RB
    return pl.pallas_call(
        _init_body,
        grid=(g,),
        in_specs=[
            pl.BlockSpec((RB, 2), lambda i: (i, 0)),
            pl.BlockSpec((RB, 16), lambda i: (i, 0)),
            pl.BlockSpec((128, 128), lambda i: (0, 0)),
            pl.BlockSpec((8, 128), lambda i: (0, 0)),
            pl.BlockSpec((128, 128), lambda i: (0, 0)),
            pl.BlockSpec((1, 128), lambda i: (0, 0)),
            pl.BlockSpec((128, 128), lambda i: (0, 0)),
            pl.BlockSpec((128, 128), lambda i: (0, 0)),
            pl.BlockSpec((128, 16), lambda i: (0, 0)),
            pl.BlockSpec((128, 16), lambda i: (0, 0)),
        ],
        out_specs=[
            pl.BlockSpec((RB, 128), lambda i: (i, 0)),
            pl.BlockSpec((RB, 128), lambda i: (i, 0)),
            pl.BlockSpec((RB, 128), lambda i: (i, 0)),
            pl.BlockSpec((RB, 16), lambda i: (i, 0)),
            pl.BlockSpec((RB, 16), lambda i: (i, 0)),
        ],
        out_shape=[
            jax.ShapeDtypeStruct((NP, 128), _f32),
            jax.ShapeDtypeStruct((NP, 128), _f32),
            jax.ShapeDtypeStruct((NP, 128), _f32),
            jax.ShapeDtypeStruct((NP, 16), _f32),
            jax.ShapeDtypeStruct((NP, 16), _f32),
        ],
    )(xp, possh, e1p, e2p, l1w, l1b2, wa, wb, fpa, fpb)


def _edge_body(gsp_ref, gdp_ref, exc_ref,
               fe2w_ref, fpk_ref, fp2_ref, ppk_ref, outh_ref, outp_ref):
    a = gsp_ref[...] + gdp_ref[...]
    ex = exc_ref[...]
    es = ex[:, 0:16]
    ed = ex[:, 16:32]
    s16 = es + ed
    d16 = es - ed
    dd = d16 * ppk_ref[4:5, :]
    dist = jnp.sqrt(jnp.sum(dd * dd, axis=1, keepdims=True) + 1e-12)
    s1 = _silu(a + dist * fpk_ref[0:1, :] + fpk_ref[1:2, :])
    m = _silu(jnp.dot(s1, fe2w_ref[...], preferred_element_type=_f32)
              + fpk_ref[2:3, :])
    e = jax.nn.sigmoid(jnp.sum(m * fpk_ref[3:4, :], axis=1, keepdims=True)
                       + ppk_ref[5:6, 0:1])
    tp = _silu(s16 * ppk_ref[3:4, :] + dist * ppk_ref[0:1, :] + ppk_ref[1:2, :])
    mp = _silu(jnp.dot(tp, fp2_ref[...], preferred_element_type=_f32)
               + ppk_ref[2:3, :])
    outh_ref[...] = e * m
    outp_ref[...] = jnp.concatenate(
        [dist * mp, jnp.zeros((mp.shape[0], 112), _f32)], axis=1)


def _tc_edge(gsp, gdp, exc, fe2w, fpk, fp2p, ppk):
    g = SLAB // BE
    return pl.pallas_call(
        _edge_body,
        grid=(g,),
        in_specs=[
            pl.BlockSpec((BE, 128), lambda i: (i, 0)),
            pl.BlockSpec((BE, 128), lambda i: (i, 0)),
            pl.BlockSpec((BE, 128), lambda i: (i, 0)),
            pl.BlockSpec((128, 128), lambda i: (0, 0)),
            pl.BlockSpec((8, 128), lambda i: (0, 0)),
            pl.BlockSpec((16, 16), lambda i: (0, 0)),
            pl.BlockSpec((8, 16), lambda i: (0, 0)),
        ],
        out_specs=[
            pl.BlockSpec((BE, 128), lambda i: (i, 0)),
            pl.BlockSpec((BE, 128), lambda i: (i, 0)),
        ],
        out_shape=[
            jax.ShapeDtypeStruct((SLAB, 128), _f32),
            jax.ShapeDtypeStruct((SLAB, 128), _f32),
        ],
    )(gsp, gdp, exc, fe2w, fpk, fp2p, ppk)


def _node1_body(h_ref, h0_ref, p16_ref, pha_ref, phb_ref, ppa_ref, ppb_ref,
                fh1a_ref, fh1b_ref, fh2_ref, hpk_ref,
                fhp1a_ref, fhp1b_ref, fhp2_ref, ppk_ref,
                wa_ref, wb_ref, fpa_ref, fpb_ref, shift_ref,
                h1_ref, p1_ref, tsp_ref, tdp_ref, tse_ref, tde_ref):
    mh = pha_ref[0] + phb_ref[0]
    mp = (ppa_ref[0][:, 0:16] + ppb_ref[0][:, 0:16]) * (1.0 / N)
    h = h_ref[...]
    u = _silu(jnp.dot(h, fh1a_ref[...], preferred_element_type=_f32)
              + jnp.dot(mh, fh1b_ref[...], preferred_element_type=_f32)
              + hpk_ref[0:1, :])
    hn = h + jnp.dot(u, fh2_ref[...], preferred_element_type=_f32) + hpk_ref[1:2, :]
    h1 = hn + h0_ref[...]
    p = p16_ref[...]
    tu = _silu(jnp.dot(p, fhp1a_ref[...], preferred_element_type=_f32)
               + jnp.dot(mp, fhp1b_ref[...], preferred_element_type=_f32)
               + ppk_ref[0:1, :])
    pn = p + jnp.dot(tu, fhp2_ref[...], preferred_element_type=_f32) + ppk_ref[1:2, :]
    psh = jnp.dot(pn, shift_ref[...], preferred_element_type=_f32)
    h1_ref[...] = h1
    p1_ref[...] = pn
    tsp_ref[...] = jnp.dot(h1, wa_ref[...], preferred_element_type=_f32)
    tdp_ref[...] = jnp.dot(h1, wb_ref[...], preferred_element_type=_f32)
    tse_ref[...] = jnp.dot(h1, fpa_ref[...], preferred_element_type=_f32) + psh
    tde_ref[...] = jnp.dot(h1, fpb_ref[...], preferred_element_type=_f32) + psh


def _tc_node1(h, h0, p16, parts_h, parts_p, fh1a, fh1b, fh2w, hpk,
              fhp1a, fhp1b, fhp2p, ppk, wa, wb, fpa, fpb, shiftm):
    g = NP // RB
    full = lambda r, c: pl.BlockSpec((r, c), lambda i: (0, 0))
    return pl.pallas_call(
        _node1_body,
        grid=(g,),
        in_specs=[
            pl.BlockSpec((RB, 128), lambda i: (i, 0)),
            pl.BlockSpec((RB, 128), lambda i: (i, 0)),
            pl.BlockSpec((RB, 16), lambda i: (i, 0)),
            pl.BlockSpec((1, RB, 128), lambda i: (0, i, 0)),
            pl.BlockSpec((1, RB, 128), lambda i: (1, i, 0)),
            pl.BlockSpec((1, RB, 128), lambda i: (0, i, 0)),
            pl.BlockSpec((1, RB, 128), lambda i: (1, i, 0)),
            full(128, 128), full(128, 128), full(128, 128), full(8, 128),
            full(16, 16), full(16, 16), full(16, 16), full(8, 16),
            full(128, 128), full(128, 128), full(128, 16), full(128, 16),
            full(16, 16),
        ],
        out_specs=[
            pl.BlockSpec((RB, 128), lambda i: (i, 0)),
            pl.BlockSpec((RB, 16), lambda i: (i, 0)),
            pl.BlockSpec((RB, 128), lambda i: (i, 0)),
            pl.BlockSpec((RB, 128), lambda i: (i, 0)),
            pl.BlockSpec((RB, 16), lambda i: (i, 0)),
            pl.BlockSpec((RB, 16), lambda i: (i, 0)),
        ],
        out_shape=[
            jax.ShapeDtypeStruct((NP, 128), _f32),
            jax.ShapeDtypeStruct((NP, 16), _f32),
            jax.ShapeDtypeStruct((NP, 128), _f32),
            jax.ShapeDtypeStruct((NP, 128), _f32),
            jax.ShapeDtypeStruct((NP, 16), _f32),
            jax.ShapeDtypeStruct((NP, 16), _f32),
        ],
    )(h, h0, p16, parts_h, parts_h, parts_p, parts_p, fh1a, fh1b, fh2w, hpk,
      fhp1a, fhp1b, fhp2p, ppk, wa, wb, fpa, fpb, shiftm)


def _node2_body(h_ref, h0_ref, pha_ref, phb_ref,
                fh1a_ref, fh1b_ref, fh2_ref, hpk_ref, out_ref):
    i = pl.program_id(0)
    mh = pha_ref[0] + phb_ref[0]
    h = h_ref[...]
    u = _silu(jnp.dot(h, fh1a_ref[...], preferred_element_type=_f32)
              + jnp.dot(mh, fh1b_ref[...], preferred_element_type=_f32)
              + hpk_ref[0:1, :])
    hn = h + jnp.dot(u, fh2_ref[...], preferred_element_type=_f32) + hpk_ref[1:2, :]
    h2 = hn + h0_ref[...]
    rows = i * RB + lax.broadcasted_iota(jnp.int32, (RB, 1), 0)
    h2m = jnp.where(rows < N, h2, 0.0)
    s = jnp.sum(h2m, axis=0, keepdims=True) * (1.0 / N)

    @pl.when(i == 0)
    def _():
        out_ref[...] = s

    @pl.when(i > 0)
    def _():
        out_ref[...] += s


def _tc_node2(h, h0, parts_h, fh1a, fh1b, fh2w, hpk):
    g = NP // RB
    full = lambda r, c: pl.BlockSpec((r, c), lambda i: (0, 0))
    return pl.pallas_call(
        _node2_body,
        grid=(g,),
        in_specs=[
            pl.BlockSpec((RB, 128), lambda i: (i, 0)),
            pl.BlockSpec((RB, 128), lambda i: (i, 0)),
            pl.BlockSpec((1, RB, 128), lambda i: (0, i, 0)),
            pl.BlockSpec((1, RB, 128), lambda i: (1, i, 0)),
            full(128, 128), full(128, 128), full(128, 128), full(8, 128),
        ],
        out_specs=pl.BlockSpec((1, 128), lambda i: (0, 0)),
        out_shape=jax.ShapeDtypeStruct((1, 128), _f32),
    )(h, h0, parts_h, parts_h, fh1a, fh1b, fh2w, hpk)


@functools.cache
def _make_sc_gather():
    mesh = plsc.VectorSubcoreMesh(core_axis_name="c", subcore_axis_name="s",
                                  num_cores=NC, num_subcores=NS)

    @functools.partial(
        pl.kernel,
        out_type=[jax.ShapeDtypeStruct((SLAB, 128), _f32),
                  jax.ShapeDtypeStruct((SLAB, 128), _f32),
                  jax.ShapeDtypeStruct((SLAB, 128), _f32)],
        mesh=mesh,
        scratch_types=[
            pltpu.VMEM((NCHS, CH), jnp.int32),
            pltpu.VMEM((NCHS, CH), jnp.int32),
            pltpu.VMEM((CH, 128), _f32),
            pltpu.VMEM((CH, 128), _f32),
            pltpu.VMEM((CH, 16), _f32),
            pltpu.VMEM((CH, 16), _f32),
            pltpu.VMEM((CH, 128), _f32),
            pltpu.VMEM((CH, 128), _f32),
            pltpu.VMEM((CH, 16), _f32),
            pltpu.VMEM((CH, 16), _f32),
            pltpu.SemaphoreType.DMA,
            pltpu.SemaphoreType.DMA,
        ],
        compiler_params=pltpu.CompilerParams(use_tc_tiling_on_sc=False),
    )
    def sc_gather(tsp_hbm, tdp_hbm, tse_hbm, tde_hbm, src_hbm, dst_hbm,
                  gsp_hbm, gdp_hbm, exc_hbm,
                  idxs_v, idxd_v,
                  b0sp, b0dp, b0se, b0de, b1sp, b1dp, b1se, b1de,
                  sem0, sem1):
        cid = lax.axis_index("c")
        sid = lax.axis_index("s")
        wid = sid * NC + cid
        pltpu.sync_copy(src_hbm.at[wid], idxs_v)
        pltpu.sync_copy(dst_hbm.at[wid], idxd_v)
        base = wid * EPWS
        tables = (tsp_hbm, tdp_hbm, tse_hbm, tde_hbm)
        idxes = (idxs_v, idxd_v, idxs_v, idxd_v)
        bufs = ((b0sp, b0dp, b0se, b0de), (b1sp, b1dp, b1se, b1de))

        def issue(j, p, sem):
            for t, b, idx in zip(tables, bufs[p], idxes):
                pltpu.async_copy(t.at[idx.at[j]], b, sem)

        def drain_write(j, p, sem):
            for t, b, idx in zip(tables, bufs[p], idxes):
                pltpu.make_async_copy(t.at[idx.at[j]], b, sem).wait()
            r = pl.ds(base + j * CH, CH)
            pltpu.sync_copy(bufs[p][0], gsp_hbm.at[r])
            pltpu.sync_copy(bufs[p][1], gdp_hbm.at[r])
            pltpu.sync_copy(bufs[p][2], exc_hbm.at[r, pl.ds(0, 16)])
            pltpu.sync_copy(bufs[p][3], exc_hbm.at[r, pl.ds(16, 16)])

        issue(0, 0, sem0)

        def body(k, carry):
            g0 = 2 * k
            issue(g0 + 1, 1, sem1)
            drain_write(g0, 0, sem0)
            issue(g0 + 2, 0, sem0)
            drain_write(g0 + 1, 1, sem1)
            return carry

        lax.fori_loop(0, (NCHS - 1) // 2, body, 0)
        drain_write(NCHS - 1, 0, sem0)

    return sc_gather


def _sc_gather(tsp, tdp, tse, tde, src2, dst2):
    return _make_sc_gather()(tsp, tdp, tse, tde, src2, dst2)


@functools.cache
def _make_sc_scatter():
    mesh = plsc.VectorSubcoreMesh(core_axis_name="c", subcore_axis_name="s",
                                  num_cores=NC, num_subcores=NS)

    @functools.partial(
        pl.kernel,
        out_type=[jax.ShapeDtypeStruct((NC, NP, 128), _f32),
                  jax.ShapeDtypeStruct((NC, NP, 128), _f32)],
        mesh=mesh,
        scratch_types=[
            pltpu.VMEM_SHARED((N, 128), _f32),
            pltpu.VMEM_SHARED((N, 16), _f32),
            pltpu.VMEM((NCHS, CH), jnp.int32),
            pltpu.VMEM((CH, 128), _f32),
            pltpu.VMEM((CH, 128), _f32),
            pltpu.VMEM((CH, 16), _f32),
            pltpu.VMEM((CH, 16), _f32),
            pltpu.SemaphoreType.DMA,
            pltpu.SemaphoreType.DMA,
        ],
        compiler_params=pltpu.CompilerParams(use_tc_tiling_on_sc=False),
    )
    def sc_scatter(ch0_hbm, ch1_hbm, ch2_hbm, ch3_hbm, ch4_hbm,
                   cp0_hbm, cp1_hbm, cp2_hbm, cp3_hbm, cp4_hbm,
                   src_hbm, zeros_hbm, parth_hbm, partp_hbm,
                   acch_sh, accp_sh, idx_v, bh0, bh1, bp0, bp1, s0, s1):
        chs = (ch0_hbm, ch1_hbm, ch2_hbm, ch3_hbm, ch4_hbm)
        cps = (cp0_hbm, cp1_hbm, cp2_hbm, cp3_hbm, cp4_hbm)
        cid = lax.axis_index("c")
        sid = lax.axis_index("s")
        wid = sid * NC + cid

        for k in range((NZC + NS - 1) // NS):
            c = sid + k * NS

            @pl.when(c < NZC)
            def _(c=c):
                r = pl.ds(c * RZ, RZ)
                pltpu.sync_copy(zeros_hbm.at[r], bh0)
                pltpu.sync_copy(bh0, acch_sh.at[r])
                pltpu.sync_copy(zeros_hbm.at[r, pl.ds(0, 16)], bp0)
                pltpu.sync_copy(bp0, accp_sh.at[r])

        plsc.subcore_barrier()
        base = wid * EPWS

        for slab in range(K):
            conh_hbm = chs[slab]
            conp_hbm = cps[slab]
            pltpu.sync_copy(src_hbm.at[slab, wid], idx_v)

            def issue(j, bh, bp, sem):
                r = pl.ds(base + j * CH, CH)
                pltpu.async_copy(conh_hbm.at[r], bh, sem)
                pltpu.async_copy(conp_hbm.at[r, pl.ds(0, 16)], bp, sem)

            def drain_scatter(j, bh, bp, sem):
                r = pl.ds(base + j * CH, CH)
                pltpu.make_async_copy(conh_hbm.at[r], bh, sem).wait()
                pltpu.make_async_copy(conp_hbm.at[r, pl.ds(0, 16)], bp, sem).wait()
                pltpu.sync_copy(bh, acch_sh.at[idx_v.at[j]], add=True)
                pltpu.sync_copy(bp, accp_sh.at[idx_v.at[j]], add=True)

            issue(0, bh0, bp0, s0)

            def body(k, carry):
                g0 = 2 * k
                issue(g0 + 1, bh1, bp1, s1)
                drain_scatter(g0, bh0, bp0, s0)
                issue(g0 + 2, bh0, bp0, s0)
                drain_scatter(g0 + 1, bh1, bp1, s1)
                return carry

            lax.fori_loop(0, (NCHS - 1) // 2, body, 0)
            drain_scatter(NCHS - 1, bh0, bp0, s0)
        plsc.subcore_barrier()

        for k in range((NZC + NS - 1) // NS):
            c = sid + k * NS

            @pl.when(c < NZC)
            def _(c=c):
                r = pl.ds(c * RZ, RZ)
                pltpu.sync_copy(acch_sh.at[r], bh0)
                pltpu.sync_copy(bh0, parth_hbm.at[cid, r])
                pltpu.sync_copy(accp_sh.at[r], bp0)
                pltpu.sync_copy(bp0, partp_hbm.at[cid, r, pl.ds(0, 16)])

    return sc_scatter


def _sc_scatter(chs, cps, src3, zeros128):
    return _make_sc_scatter()(*chs, *cps, src3, zeros128)


def _conv_consts(p):
    fpk = jnp.zeros((8, 128), _f32)
    fpk = fpk.at[0].set(p["fe1_w"][256])
    fpk = fpk.at[1].set(p["fe1_b"])
    fpk = fpk.at[2].set(p["fe2_b"])
    fpk = fpk.at[3].set(p["inf_w"][:, 0])
    ppk = jnp.zeros((8, 16), _f32)
    ppk = ppk.at[0, 0:3].set(p["fp1_w"][256])
    ppk = ppk.at[1, 0:3].set(p["fp1_b"])
    ppk = ppk.at[2, 0:3].set(p["fp2_b"])
    ppk = ppk.at[3, 0:3].set(1.0)
    ppk = ppk.at[4, 3:6].set(1.0)
    ppk = ppk.at[5, 0].set(p["inf_b"][0])
    fp2p = jnp.zeros((16, 16), _f32).at[0:3, 0:3].set(p["fp2_w"])
    wa = p["fe1_w"][0:128]
    wb = p["fe1_w"][128:256]
    fpa = jnp.pad(p["fp1_w"][0:128], ((0, 0), (0, 13)))
    fpb = jnp.pad(p["fp1_w"][128:256], ((0, 0), (0, 13)))
    hpk = jnp.zeros((8, 128), _f32)
    hpk = hpk.at[0].set(p["fh1_b"])
    hpk = hpk.at[1].set(p["fh2_b"])
    fh1a = p["fh1_w"][0:128]
    fh1b = p["fh1_w"][128:256]
    fhp1a = jnp.zeros((16, 16), _f32).at[0:3, 0:3].set(p["fhp1_w"][0:3])
    fhp1b = jnp.zeros((16, 16), _f32).at[0:3, 0:3].set(p["fhp1_w"][3:6])
    fhp2p = jnp.zeros((16, 16), _f32).at[0:3, 0:3].set(p["fhp2_w"])
    ppku = jnp.zeros((8, 16), _f32)
    ppku = ppku.at[0, 0:3].set(p["fhp1_b"])
    ppku = ppku.at[1, 0:3].set(p["fhp2_b"])
    return dict(fpk=fpk, ppk=ppk, fp2p=fp2p, wa=wa, wb=wb, fpa=fpa, fpb=fpb,
                hpk=hpk, fh1a=fh1a, fh1b=fh1b, fhp1a=fhp1a, fhp1b=fhp1b,
                fhp2p=fhp2p, ppku=ppku, fe2w=p["fe2_w"])


def kernel(x, edge_index, pos, emb1, emb2, l1_w, l1_b, params):
    xp = jnp.pad(x.astype(jnp.int32), ((0, NP - N), (0, 0)))
    pos = pos.astype(_f32)
    possh = jnp.pad(pos, ((0, NP - N), (3, 10)))
    p16 = jnp.pad(pos, ((0, NP - N), (0, 13)))
    src3 = edge_index[:, 0].astype(jnp.int32).reshape(K, NW, NCHS, CH)
    dst3 = edge_index[:, 1].astype(jnp.int32).reshape(K, NW, NCHS, CH)
    zeros128 = jnp.zeros((N, 128), _f32)
    e1p = jnp.pad(emb1, ((0, 128 - emb1.shape[0]), (0, 0)))
    e2p = jnp.pad(emb2, ((0, 8 - emb2.shape[0]), (0, 0)))
    l1b2 = l1_b[None, :]
    shiftm = jnp.zeros((16, 16), _f32).at[0, 3].set(1.0).at[1, 4].set(1.0).at[2, 5].set(1.0)

    c1 = _conv_consts(params["c1"])
    c2 = _conv_consts(params["c2"])

    def conv_edges(tsp, tdp, tse, tde, cc):
        chs, cps = [], []
        for k in range(K):
            gsp, gdp, exc = _sc_gather(tsp, tdp, tse, tde, src3[k], dst3[k])
            ch, cp = _tc_edge(gsp, gdp, exc,
                              cc["fe2w"], cc["fpk"], cc["fp2p"], cc["ppk"])
            chs.append(ch)
            cps.append(cp)
        return _sc_scatter(chs, cps, src3, zeros128)

    h0, t1sp, t1dp, t1se, t1de = _tc_init(xp, possh, e1p, e2p, l1_w, l1b2,
                                          c1["wa"], c1["wb"], c1["fpa"], c1["fpb"])
    ph1, pp1 = conv_edges(t1sp, t1dp, t1se, t1de, c1)
    h1, p16_1, t2sp, t2dp, t2se, t2de = _tc_node1(
        h0, h0, p16, ph1, pp1,
        c1["fh1a"], c1["fh1b"], params["c1"]["fh2_w"], c1["hpk"],
        c1["fhp1a"], c1["fhp1b"], c1["fhp2p"], c1["ppku"],
        c2["wa"], c2["wb"], c2["fpa"], c2["fpb"], shiftm)
    ph2, _pp2 = conv_edges(t2sp, t2dp, t2se, t2de, c2)
    out = _tc_node2(h1, h0, ph2,
                    c2["fh1a"], c2["fh1b"], params["c2"]["fh2_w"], c2["hpk"])
    return out


# speedup vs baseline: 2.8787x; 1.0792x over previous
import functools

import jax
import jax.numpy as jnp
from jax import lax
from jax.experimental import pallas as pl
from jax.experimental.pallas import tpu as pltpu
from jax.experimental.pallas import tpu_sc as plsc

N = 10000
NP = 10240
E = 320000
H = 128
RB = 1024
BE = 4000
NC = 2
NS = 16
NW = NC * NS
EPW = E // NW
CH = 80
NCH = EPW // CH
K = 5
SLAB = E // K
EPWS = SLAB // NW
NCHS = EPWS // CH
RZ = 80
NZC = N // RZ

_f32 = jnp.float32


def _silu(v):
    return v * jax.nn.sigmoid(v)


def _init_body(x_ref, possh_ref, e1_ref, e2_ref, l1w_ref, l1b_ref,
               wa_ref, wb_ref, fpa_ref, fpb_ref,
               h0_ref, tsp_ref, tdp_ref, tse_ref, tde_ref):
    xb = x_ref[...]
    oh1 = (xb[:, 0:1] == lax.broadcasted_iota(jnp.int32, (1, 128), 1)
           ).astype(_f32)
    oh2 = (xb[:, 1:2] == lax.broadcasted_iota(jnp.int32, (1, 8), 1)
           ).astype(_f32)
    hemb = (jnp.dot(oh1, e1_ref[...], preferred_element_type=_f32)
            + jnp.dot(oh2, e2_ref[...], preferred_element_type=_f32))
    h0 = jax.nn.relu(jnp.dot(hemb, l1w_ref[...], preferred_element_type=_f32)
                     + l1b_ref[...])
    possh = possh_ref[...]
    h0_ref[...] = h0
    tsp_ref[...] = jnp.dot(h0, wa_ref[...], preferred_element_type=_f32)
    tdp_ref[...] = jnp.dot(h0, wb_ref[...], preferred_element_type=_f32)
    tse_ref[...] = jnp.dot(h0, fpa_ref[...], preferred_element_type=_f32) + possh
    tde_ref[...] = jnp.dot(h0, fpb_ref[...], preferred_element_type=_f32) + possh


def _tc_init(xp, possh, e1p, e2p, l1w, l1b2, wa, wb, fpa, fpb):
    g = NP // RB
    return pl.pallas_call(
        _init_body,
        grid=(g,),
        in_specs=[
            pl.BlockSpec((RB, 2), lambda i: (i, 0)),
            pl.BlockSpec((RB, 16), lambda i: (i, 0)),
            pl.BlockSpec((128, 128), lambda i: (0, 0)),
            pl.BlockSpec((8, 128), lambda i: (0, 0)),
            pl.BlockSpec((128, 128), lambda i: (0, 0)),
            pl.BlockSpec((1, 128), lambda i: (0, 0)),
            pl.BlockSpec((128, 128), lambda i: (0, 0)),
            pl.BlockSpec((128, 128), lambda i: (0, 0)),
            pl.BlockSpec((128, 16), lambda i: (0, 0)),
            pl.BlockSpec((128, 16), lambda i: (0, 0)),
        ],
        out_specs=[
            pl.BlockSpec((RB, 128), lambda i: (i, 0)),
            pl.BlockSpec((RB, 128), lambda i: (i, 0)),
            pl.BlockSpec((RB, 128), lambda i: (i, 0)),
            pl.BlockSpec((RB, 16), lambda i: (i, 0)),
            pl.BlockSpec((RB, 16), lambda i: (i, 0)),
        ],
        out_shape=[
            jax.ShapeDtypeStruct((NP, 128), _f32),
            jax.ShapeDtypeStruct((NP, 128), _f32),
            jax.ShapeDtypeStruct((NP, 128), _f32),
            jax.ShapeDtypeStruct((NP, 16), _f32),
            jax.ShapeDtypeStruct((NP, 16), _f32),
        ],
    )(xp, possh, e1p, e2p, l1w, l1b2, wa, wb, fpa, fpb)


def _edge_body(with_pos, gsp_ref, gdp_ref, exc_ref,
               fe2w_ref, fpk_ref, fp2_ref, ppk_ref, outh_ref, outp_ref=None):
    a = gsp_ref[...] + gdp_ref[...]
    ex = exc_ref[...]
    es = ex[:, 0:16]
    ed = ex[:, 16:32]
    d16 = es - ed
    dd = d16 * ppk_ref[4:5, :]
    dist = jnp.sqrt(jnp.sum(dd * dd, axis=1, keepdims=True) + 1e-12)
    s1 = _silu(a + dist * fpk_ref[0:1, :] + fpk_ref[1:2, :])
    m = _silu(jnp.dot(s1, fe2w_ref[...], preferred_element_type=_f32)
              + fpk_ref[2:3, :])
    e = jax.nn.sigmoid(jnp.sum(m * fpk_ref[3:4, :], axis=1, keepdims=True)
                       + ppk_ref[5:6, 0:1])
    outh_ref[...] = e * m
    if with_pos:
        s16 = es + ed
        tp = _silu(s16 * ppk_ref[3:4, :] + dist * ppk_ref[0:1, :]
                   + ppk_ref[1:2, :])
        mp = _silu(jnp.dot(tp, fp2_ref[...], preferred_element_type=_f32)
                   + ppk_ref[2:3, :])
        outp_ref[...] = jnp.concatenate(
            [dist * mp, jnp.zeros((mp.shape[0], 112), _f32)], axis=1)


def _tc_edge(gsp, gdp, exc, fe2w, fpk, fp2p, ppk, with_pos=True):
    g = SLAB // BE
    n_out = 2 if with_pos else 1
    out = pl.pallas_call(
        functools.partial(_edge_body, with_pos),
        grid=(g,),
        in_specs=[
            pl.BlockSpec((BE, 128), lambda i: (i, 0)),
            pl.BlockSpec((BE, 128), lambda i: (i, 0)),
            pl.BlockSpec((BE, 128), lambda i: (i, 0)),
            pl.BlockSpec((128, 128), lambda i: (0, 0)),
            pl.BlockSpec((8, 128), lambda i: (0, 0)),
            pl.BlockSpec((16, 16), lambda i: (0, 0)),
            pl.BlockSpec((8, 16), lambda i: (0, 0)),
        ],
        out_specs=[pl.BlockSpec((BE, 128), lambda i: (i, 0))] * n_out,
        out_shape=[jax.ShapeDtypeStruct((SLAB, 128), _f32)] * n_out,
    )(gsp, gdp, exc, fe2w, fpk, fp2p, ppk)
    return out if with_pos else (out[0], None)


def _node1_body(h_ref, h0_ref, p16_ref, pha_ref, phb_ref, ppa_ref, ppb_ref,
                fh1a_ref, fh1b_ref, fh2_ref, hpk_ref,
                fhp1a_ref, fhp1b_ref, fhp2_ref, ppk_ref,
                wa_ref, wb_ref, fpa_ref, fpb_ref, shift_ref,
                h1_ref, p1_ref, tsp_ref, tdp_ref, tse_ref, tde_ref):
    mh = pha_ref[0] + phb_ref[0]
    mp = (ppa_ref[0][:, 0:16] + ppb_ref[0][:, 0:16]) * (1.0 / N)
    h = h_ref[...]
    u = _silu(jnp.dot(h, fh1a_ref[...], preferred_element_type=_f32)
              + jnp.dot(mh, fh1b_ref[...], preferred_element_type=_f32)
              + hpk_ref[0:1, :])
    hn = h + jnp.dot(u, fh2_ref[...], preferred_element_type=_f32) + hpk_ref[1:2, :]
    h1 = hn + h0_ref[...]
    p = p16_ref[...]
    tu = _silu(jnp.dot(p, fhp1a_ref[...], preferred_element_type=_f32)
               + jnp.dot(mp, fhp1b_ref[...], preferred_element_type=_f32)
               + ppk_ref[0:1, :])
    pn = p + jnp.dot(tu, fhp2_ref[...], preferred_element_type=_f32) + ppk_ref[1:2, :]
    psh = jnp.dot(pn, shift_ref[...], preferred_element_type=_f32)
    h1_ref[...] = h1
    p1_ref[...] = pn
    tsp_ref[...] = jnp.dot(h1, wa_ref[...], preferred_element_type=_f32)
    tdp_ref[...] = jnp.dot(h1, wb_ref[...], preferred_element_type=_f32)
    tse_ref[...] = jnp.dot(h1, fpa_ref[...], preferred_element_type=_f32) + psh
    tde_ref[...] = jnp.dot(h1, fpb_ref[...], preferred_element_type=_f32) + psh


def _tc_node1(h, h0, p16, parts_h, parts_p, fh1a, fh1b, fh2w, hpk,
              fhp1a, fhp1b, fhp2p, ppk, wa, wb, fpa, fpb, shiftm):
    g = NP // RB
    full = lambda r, c: pl.BlockSpec((r, c), lambda i: (0, 0))
    return pl.pallas_call(
        _node1_body,
        grid=(g,),
        in_specs=[
            pl.BlockSpec((RB, 128), lambda i: (i, 0)),
            pl.BlockSpec((RB, 128), lambda i: (i, 0)),
            pl.BlockSpec((RB, 16), lambda i: (i, 0)),
            pl.BlockSpec((1, RB, 128), lambda i: (0, i, 0)),
            pl.BlockSpec((1, RB, 128), lambda i: (1, i, 0)),
            pl.BlockSpec((1, RB, 128), lambda i: (0, i, 0)),
            pl.BlockSpec((1, RB, 128), lambda i: (1, i, 0)),
            full(128, 128), full(128, 128), full(128, 128), full(8, 128),
            full(16, 16), full(16, 16), full(16, 16), full(8, 16),
            full(128, 128), full(128, 128), full(128, 16), full(128, 16),
            full(16, 16),
        ],
        out_specs=[
            pl.BlockSpec((RB, 128), lambda i: (i, 0)),
            pl.BlockSpec((RB, 16), lambda i: (i, 0)),
            pl.BlockSpec((RB, 128), lambda i: (i, 0)),
            pl.BlockSpec((RB, 128), lambda i: (i, 0)),
            pl.BlockSpec((RB, 16), lambda i: (i, 0)),
            pl.BlockSpec((RB, 16), lambda i: (i, 0)),
        ],
        out_shape=[
            jax.ShapeDtypeStruct((NP, 128), _f32),
            jax.ShapeDtypeStruct((NP, 16), _f32),
            jax.ShapeDtypeStruct((NP, 128), _f32),
            jax.ShapeDtypeStruct((NP, 128), _f32),
            jax.ShapeDtypeStruct((NP, 16), _f32),
            jax.ShapeDtypeStruct((NP, 16), _f32),
        ],
    )(h, h0, p16, parts_h, parts_h, parts_p, parts_p, fh1a, fh1b, fh2w, hpk,
      fhp1a, fhp1b, fhp2p, ppk, wa, wb, fpa, fpb, shiftm)


def _node2_body(h_ref, h0_ref, pha_ref, phb_ref,
                fh1a_ref, fh1b_ref, fh2_ref, hpk_ref, out_ref):
    i = pl.program_id(0)
    mh = pha_ref[0] + phb_ref[0]
    h = h_ref[...]
    u = _silu(jnp.dot(h, fh1a_ref[...], preferred_element_type=_f32)
              + jnp.dot(mh, fh1b_ref[...], preferred_element_type=_f32)
              + hpk_ref[0:1, :])
    hn = h + jnp.dot(u, fh2_ref[...], preferred_element_type=_f32) + hpk_ref[1:2, :]
    h2 = hn + h0_ref[...]
    rows = i * RB + lax.broadcasted_iota(jnp.int32, (RB, 1), 0)
    h2m = jnp.where(rows < N, h2, 0.0)
    s = jnp.sum(h2m, axis=0, keepdims=True) * (1.0 / N)

    @pl.when(i == 0)
    def _():
        out_ref[...] = s

    @pl.when(i > 0)
    def _():
        out_ref[...] += s


def _tc_node2(h, h0, parts_h, fh1a, fh1b, fh2w, hpk):
    g = NP // RB
    full = lambda r, c: pl.BlockSpec((r, c), lambda i: (0, 0))
    return pl.pallas_call(
        _node2_body,
        grid=(g,),
        in_specs=[
            pl.BlockSpec((RB, 128), lambda i: (i, 0)),
            pl.BlockSpec((RB, 128), lambda i: (i, 0)),
            pl.BlockSpec((1, RB, 128), lambda i: (0, i, 0)),
            pl.BlockSpec((1, RB, 128), lambda i: (1, i, 0)),
            full(128, 128), full(128, 128), full(128, 128), full(8, 128),
        ],
        out_specs=pl.BlockSpec((1, 128), lambda i: (0, 0)),
        out_shape=jax.ShapeDtypeStruct((1, 128), _f32),
    )(h, h0, parts_h, parts_h, fh1a, fh1b, fh2w, hpk)


@functools.cache
def _make_sc_gather():
    mesh = plsc.VectorSubcoreMesh(core_axis_name="c", subcore_axis_name="s",
                                  num_cores=NC, num_subcores=NS)

    @functools.partial(
        pl.kernel,
        out_type=[jax.ShapeDtypeStruct((SLAB, 128), _f32),
                  jax.ShapeDtypeStruct((SLAB, 128), _f32),
                  jax.ShapeDtypeStruct((SLAB, 128), _f32)],
        mesh=mesh,
        scratch_types=[
            pltpu.VMEM((NCHS, CH), jnp.int32),
            pltpu.VMEM((NCHS, CH), jnp.int32),
            pltpu.VMEM((CH, 128), _f32),
            pltpu.VMEM((CH, 128), _f32),
            pltpu.VMEM((CH, 16), _f32),
            pltpu.VMEM((CH, 16), _f32),
            pltpu.VMEM((CH, 128), _f32),
            pltpu.VMEM((CH, 128), _f32),
            pltpu.VMEM((CH, 16), _f32),
            pltpu.VMEM((CH, 16), _f32),
            pltpu.SemaphoreType.DMA,
            pltpu.SemaphoreType.DMA,
        ],
        compiler_params=pltpu.CompilerParams(use_tc_tiling_on_sc=False),
    )
    def sc_gather(tsp_hbm, tdp_hbm, tse_hbm, tde_hbm, src_hbm, dst_hbm,
                  gsp_hbm, gdp_hbm, exc_hbm,
                  idxs_v, idxd_v,
                  b0sp, b0dp, b0se, b0de, b1sp, b1dp, b1se, b1de,
                  sem0, sem1):
        cid = lax.axis_index("c")
        sid = lax.axis_index("s")
        wid = sid * NC + cid
        pltpu.sync_copy(src_hbm.at[wid], idxs_v)
        pltpu.sync_copy(dst_hbm.at[wid], idxd_v)
        base = wid * EPWS
        tables = (tsp_hbm, tdp_hbm, tse_hbm, tde_hbm)
        idxes = (idxs_v, idxd_v, idxs_v, idxd_v)
        bufs = ((b0sp, b0dp, b0se, b0de), (b1sp, b1dp, b1se, b1de))

        def issue(j, p, sem):
            for t, b, idx in zip(tables, bufs[p], idxes):
                pltpu.async_copy(t.at[idx.at[j]], b, sem)

        def drain_write(j, p, sem):
            for t, b, idx in zip(tables, bufs[p], idxes):
                pltpu.make_async_copy(t.at[idx.at[j]], b, sem).wait()
            r = pl.ds(base + j * CH, CH)
            pltpu.sync_copy(bufs[p][0], gsp_hbm.at[r])
            pltpu.sync_copy(bufs[p][1], gdp_hbm.at[r])
            pltpu.sync_copy(bufs[p][2], exc_hbm.at[r, pl.ds(0, 16)])
            pltpu.sync_copy(bufs[p][3], exc_hbm.at[r, pl.ds(16, 16)])

        issue(0, 0, sem0)

        def body(k, carry):
            g0 = 2 * k
            issue(g0 + 1, 1, sem1)
            drain_write(g0, 0, sem0)
            issue(g0 + 2, 0, sem0)
            drain_write(g0 + 1, 1, sem1)
            return carry

        lax.fori_loop(0, (NCHS - 1) // 2, body, 0)
        drain_write(NCHS - 1, 0, sem0)

    return sc_gather


def _sc_gather(tsp, tdp, tse, tde, src2, dst2):
    return _make_sc_gather()(tsp, tdp, tse, tde, src2, dst2)


@functools.cache
def _make_sc_scatter(with_pos):
    mesh = plsc.VectorSubcoreMesh(core_axis_name="c", subcore_axis_name="s",
                                  num_cores=NC, num_subcores=NS)
    n_out = 2 if with_pos else 1
    scratch = [pltpu.VMEM_SHARED((N, 128), _f32)]
    if with_pos:
        scratch.append(pltpu.VMEM_SHARED((N, 16), _f32))
    scratch.append(pltpu.VMEM((NCHS, CH), jnp.int32))
    scratch += [pltpu.VMEM((CH, 128), _f32)] * 2
    if with_pos:
        scratch += [pltpu.VMEM((CH, 16), _f32)] * 2
    scratch += [pltpu.SemaphoreType.DMA] * 2

    @functools.partial(
        pl.kernel,
        out_type=[jax.ShapeDtypeStruct((NC, NP, 128), _f32)] * n_out,
        mesh=mesh,
        scratch_types=scratch,
        compiler_params=pltpu.CompilerParams(use_tc_tiling_on_sc=False),
    )
    def sc_scatter(*refs):
        it = iter(refs)
        chs = [next(it) for _ in range(K)]
        cps = [next(it) for _ in range(K)] if with_pos else [None] * K
        src_hbm = next(it)
        zeros_hbm = next(it)
        parth_hbm = next(it)
        partp_hbm = next(it) if with_pos else None
        acch_sh = next(it)
        accp_sh = next(it) if with_pos else None
        idx_v = next(it)
        bh0 = next(it)
        bh1 = next(it)
        bp0 = next(it) if with_pos else None
        bp1 = next(it) if with_pos else None
        s0 = next(it)
        s1 = next(it)
        cid = lax.axis_index("c")
        sid = lax.axis_index("s")
        wid = sid * NC + cid

        for k in range((NZC + NS - 1) // NS):
            c = sid + k * NS

            @pl.when(c < NZC)
            def _(c=c):
                r = pl.ds(c * RZ, RZ)
                pltpu.sync_copy(zeros_hbm.at[r], bh0)
                pltpu.sync_copy(bh0, acch_sh.at[r])
                if with_pos:
                    pltpu.sync_copy(zeros_hbm.at[r, pl.ds(0, 16)], bp0)
                    pltpu.sync_copy(bp0, accp_sh.at[r])

        plsc.subcore_barrier()
        base = wid * EPWS

        for slab in range(K):
            conh_hbm = chs[slab]
            conp_hbm = cps[slab]
            pltpu.sync_copy(src_hbm.at[slab, wid], idx_v)

            def issue(j, bh, bp, sem):
                r = pl.ds(base + j * CH, CH)
                pltpu.async_copy(conh_hbm.at[r], bh, sem)
                if with_pos:
                    pltpu.async_copy(conp_hbm.at[r, pl.ds(0, 16)], bp, sem)

            def drain_scatter(j, bh, bp, sem):
                r = pl.ds(base + j * CH, CH)
                pltpu.make_async_copy(conh_hbm.at[r], bh, sem).wait()
                if with_pos:
                    pltpu.make_async_copy(
                        conp_hbm.at[r, pl.ds(0, 16)], bp, sem).wait()
                pltpu.sync_copy(bh, acch_sh.at[idx_v.at[j]], add=True)
                if with_pos:
                    pltpu.sync_copy(bp, accp_sh.at[idx_v.at[j]], add=True)

            issue(0, bh0, bp0, s0)

            def body(k, carry):
                g0 = 2 * k
                issue(g0 + 1, bh1, bp1, s1)
                drain_scatter(g0, bh0, bp0, s0)
                issue(g0 + 2, bh0, bp0, s0)
                drain_scatter(g0 + 1, bh1, bp1, s1)
                return carry

            lax.fori_loop(0, (NCHS - 1) // 2, body, 0)
            drain_scatter(NCHS - 1, bh0, bp0, s0)
        plsc.subcore_barrier()

        for k in range((NZC + NS - 1) // NS):
            c = sid + k * NS

            @pl.when(c < NZC)
            def _(c=c):
                r = pl.ds(c * RZ, RZ)
                pltpu.sync_copy(acch_sh.at[r], bh0)
                pltpu.sync_copy(bh0, parth_hbm.at[cid, r])
                if with_pos:
                    pltpu.sync_copy(accp_sh.at[r], bp0)
                    pltpu.sync_copy(bp0, partp_hbm.at[cid, r, pl.ds(0, 16)])

    return sc_scatter


def _sc_scatter(chs, cps, src3, zeros128, with_pos=True):
    if with_pos:
        return _make_sc_scatter(True)(*chs, *cps, src3, zeros128)
    res = _make_sc_scatter(False)(*chs, src3, zeros128)
    return res[0] if isinstance(res, (list, tuple)) else res


def _conv_consts(p):
    fpk = jnp.zeros((8, 128), _f32)
    fpk = fpk.at[0].set(p["fe1_w"][256])
    fpk = fpk.at[1].set(p["fe1_b"])
    fpk = fpk.at[2].set(p["fe2_b"])
    fpk = fpk.at[3].set(p["inf_w"][:, 0])
    ppk = jnp.zeros((8, 16), _f32)
    ppk = ppk.at[0, 0:3].set(p["fp1_w"][256])
    ppk = ppk.at[1, 0:3].set(p["fp1_b"])
    ppk = ppk.at[2, 0:3].set(p["fp2_b"])
    ppk = ppk.at[3, 0:3].set(1.0)
    ppk = ppk.at[4, 3:6].set(1.0)
    ppk = ppk.at[5, 0].set(p["inf_b"][0])
    fp2p = jnp.zeros((16, 16), _f32).at[0:3, 0:3].set(p["fp2_w"])
    wa = p["fe1_w"][0:128]
    wb = p["fe1_w"][128:256]
    fpa = jnp.pad(p["fp1_w"][0:128], ((0, 0), (0, 13)))
    fpb = jnp.pad(p["fp1_w"][128:256], ((0, 0), (0, 13)))
    hpk = jnp.zeros((8, 128), _f32)
    hpk = hpk.at[0].set(p["fh1_b"])
    hpk = hpk.at[1].set(p["fh2_b"])
    fh1a = p["fh1_w"][0:128]
    fh1b = p["fh1_w"][128:256]
    fhp1a = jnp.zeros((16, 16), _f32).at[0:3, 0:3].set(p["fhp1_w"][0:3])
    fhp1b = jnp.zeros((16, 16), _f32).at[0:3, 0:3].set(p["fhp1_w"][3:6])
    fhp2p = jnp.zeros((16, 16), _f32).at[0:3, 0:3].set(p["fhp2_w"])
    ppku = jnp.zeros((8, 16), _f32)
    ppku = ppku.at[0, 0:3].set(p["fhp1_b"])
    ppku = ppku.at[1, 0:3].set(p["fhp2_b"])
    return dict(fpk=fpk, ppk=ppk, fp2p=fp2p, wa=wa, wb=wb, fpa=fpa, fpb=fpb,
                hpk=hpk, fh1a=fh1a, fh1b=fh1b, fhp1a=fhp1a, fhp1b=fhp1b,
                fhp2p=fhp2p, ppku=ppku, fe2w=p["fe2_w"])


def kernel(x, edge_index, pos, emb1, emb2, l1_w, l1_b, params):
    xp = jnp.pad(x.astype(jnp.int32), ((0, NP - N), (0, 0)))
    pos = pos.astype(_f32)
    possh = jnp.pad(pos, ((0, NP - N), (3, 10)))
    p16 = jnp.pad(pos, ((0, NP - N), (0, 13)))
    src3 = edge_index[:, 0].astype(jnp.int32).reshape(K, NW, NCHS, CH)
    dst3 = edge_index[:, 1].astype(jnp.int32).reshape(K, NW, NCHS, CH)
    zeros128 = jnp.zeros((N, 128), _f32)
    e1p = jnp.pad(emb1, ((0, 128 - emb1.shape[0]), (0, 0)))
    e2p = jnp.pad(emb2, ((0, 8 - emb2.shape[0]), (0, 0)))
    l1b2 = l1_b[None, :]
    shiftm = jnp.zeros((16, 16), _f32).at[0, 3].set(1.0).at[1, 4].set(1.0).at[2, 5].set(1.0)

    c1 = _conv_consts(params["c1"])
    c2 = _conv_consts(params["c2"])

    def conv_edges(tsp, tdp, tse, tde, cc, with_pos=True):
        chs, cps = [], []
        for k in range(K):
            gsp, gdp, exc = _sc_gather(tsp, tdp, tse, tde, src3[k], dst3[k])
            ch, cp = _tc_edge(gsp, gdp, exc,
                              cc["fe2w"], cc["fpk"], cc["fp2p"], cc["ppk"],
                              with_pos=with_pos)
            chs.append(ch)
            cps.append(cp)
        return _sc_scatter(chs, cps, src3, zeros128, with_pos=with_pos)

    h0, t1sp, t1dp, t1se, t1de = _tc_init(xp, possh, e1p, e2p, l1_w, l1b2,
                                          c1["wa"], c1["wb"], c1["fpa"], c1["fpb"])
    ph1, pp1 = conv_edges(t1sp, t1dp, t1se, t1de, c1)
    h1, p16_1, t2sp, t2dp, t2se, t2de = _tc_node1(
        h0, h0, p16, ph1, pp1,
        c1["fh1a"], c1["fh1b"], params["c1"]["fh2_w"], c1["hpk"],
        c1["fhp1a"], c1["fhp1b"], c1["fhp2p"], c1["ppku"],
        c2["wa"], c2["wb"], c2["fpa"], c2["fpb"], shiftm)
    ph2 = conv_edges(t2sp, t2dp, t2se, t2de, c2, with_pos=False)
    out = _tc_node2(h1, h0, ph2,
                    c2["fh1a"], c2["fh1b"], params["c2"]["fh2_w"], c2["hpk"])
    return out


# speedup vs baseline: 3.0178x; 1.0484x over previous
import functools

import jax
import jax.numpy as jnp
from jax import lax
from jax.experimental import pallas as pl
from jax.experimental.pallas import tpu as pltpu
from jax.experimental.pallas import tpu_sc as plsc

N = 10000
NP = 10240
E = 320000
H = 128
RB = 1024
BE = 4000
NC = 2
NS = 16
NW = NC * NS
EPW = E // NW
CH = 80
NCH = EPW // CH
K = 5
SLAB = E // K
EPWS = SLAB // NW
NCHS = EPWS // CH
RZ = 80
NZC = N // RZ

_f32 = jnp.float32


def _silu(v):
    return v * jax.nn.sigmoid(v)


def _init_body(x_ref, possh_ref, e1_ref, e2_ref, l1w_ref, l1b_ref,
               wa_ref, wb_ref, fpa_ref, fpb_ref,
               h0_ref, tsp_ref, tdp_ref, tse_ref, tde_ref):
    xb = x_ref[...]
    oh1 = (xb[:, 0:1] == lax.broadcasted_iota(jnp.int32, (1, 128), 1)
           ).astype(_f32)
    oh2 = (xb[:, 1:2] == lax.broadcasted_iota(jnp.int32, (1, 8), 1)
           ).astype(_f32)
    hemb = (jnp.dot(oh1, e1_ref[...], preferred_element_type=_f32)
            + jnp.dot(oh2, e2_ref[...], preferred_element_type=_f32))
    h0 = jax.nn.relu(jnp.dot(hemb, l1w_ref[...], preferred_element_type=_f32)
                     + l1b_ref[...])
    possh = possh_ref[...]
    h0_ref[...] = h0
    tsp_ref[...] = jnp.dot(h0, wa_ref[...], preferred_element_type=_f32)
    tdp_ref[...] = jnp.dot(h0, wb_ref[...], preferred_element_type=_f32)
    tse_ref[...] = jnp.dot(h0, fpa_ref[...], preferred_element_type=_f32) + possh
    tde_ref[...] = jnp.dot(h0, fpb_ref[...], preferred_element_type=_f32) + possh


def _tc_init(xp, possh, e1p, e2p, l1w, l1b2, wa, wb, fpa, fpb):
    g = NP // RB
    return pl.pallas_call(
        _init_body,
        grid=(g,),
        in_specs=[
            pl.BlockSpec((RB, 2), lambda i: (i, 0)),
            pl.BlockSpec((RB, 16), lambda i: (i, 0)),
            pl.BlockSpec((128, 128), lambda i: (0, 0)),
            pl.BlockSpec((8, 128), lambda i: (0, 0)),
            pl.BlockSpec((128, 128), lambda i: (0, 0)),
            pl.BlockSpec((1, 128), lambda i: (0, 0)),
            pl.BlockSpec((128, 128), lambda i: (0, 0)),
            pl.BlockSpec((128, 128), lambda i: (0, 0)),
            pl.BlockSpec((128, 16), lambda i: (0, 0)),
            pl.BlockSpec((128, 16), lambda i: (0, 0)),
        ],
        out_specs=[
            pl.BlockSpec((RB, 128), lambda i: (i, 0)),
            pl.BlockSpec((RB, 128), lambda i: (i, 0)),
            pl.BlockSpec((RB, 128), lambda i: (i, 0)),
            pl.BlockSpec((RB, 16), lambda i: (i, 0)),
            pl.BlockSpec((RB, 16), lambda i: (i, 0)),
        ],
        out_shape=[
            jax.ShapeDtypeStruct((NP, 128), _f32),
            jax.ShapeDtypeStruct((NP, 128), _f32),
            jax.ShapeDtypeStruct((NP, 128), _f32),
            jax.ShapeDtypeStruct((NP, 16), _f32),
            jax.ShapeDtypeStruct((NP, 16), _f32),
        ],
    )(xp, possh, e1p, e2p, l1w, l1b2, wa, wb, fpa, fpb)


def _edge_body(with_pos, gsp_ref, gdp_ref, exc_ref,
               fe2w_ref, fpk_ref, fp2_ref, ppk_ref, outh_ref, outp_ref=None):
    a = gsp_ref[...] + gdp_ref[...]
    ex = exc_ref[...]
    es = ex[:, 0:16]
    ed = ex[:, 16:32]
    d16 = es - ed
    dd = d16 * ppk_ref[4:5, :]
    dist = jnp.sqrt(jnp.sum(dd * dd, axis=1, keepdims=True) + 1e-12)
    s1 = _silu(a + dist * fpk_ref[0:1, :] + fpk_ref[1:2, :])
    m = _silu(jnp.dot(s1, fe2w_ref[...], preferred_element_type=_f32)
              + fpk_ref[2:3, :])
    e = jax.nn.sigmoid(jnp.sum(m * fpk_ref[3:4, :], axis=1, keepdims=True)
                       + ppk_ref[5:6, 0:1])
    outh_ref[...] = e * m
    if with_pos:
        s16 = es + ed
        tp = _silu(s16 * ppk_ref[3:4, :] + dist * ppk_ref[0:1, :]
                   + ppk_ref[1:2, :])
        mp = _silu(jnp.dot(tp, fp2_ref[...], preferred_element_type=_f32)
                   + ppk_ref[2:3, :])
        outp_ref[...] = jnp.concatenate(
            [dist * mp, jnp.zeros((mp.shape[0], 112), _f32)], axis=1)


def _tc_edge(gsp, gdp, exc, fe2w, fpk, fp2p, ppk, with_pos=True):
    g = SLAB // BE
    n_out = 2 if with_pos else 1
    out = pl.pallas_call(
        functools.partial(_edge_body, with_pos),
        grid=(g,),
        in_specs=[
            pl.BlockSpec((BE, 128), lambda i: (i, 0)),
            pl.BlockSpec((BE, 128), lambda i: (i, 0)),
            pl.BlockSpec((BE, 128), lambda i: (i, 0)),
            pl.BlockSpec((128, 128), lambda i: (0, 0)),
            pl.BlockSpec((8, 128), lambda i: (0, 0)),
            pl.BlockSpec((16, 16), lambda i: (0, 0)),
            pl.BlockSpec((8, 16), lambda i: (0, 0)),
        ],
        out_specs=[pl.BlockSpec((BE, 128), lambda i: (i, 0))] * n_out,
        out_shape=[jax.ShapeDtypeStruct((SLAB, 128), _f32)] * n_out,
    )(gsp, gdp, exc, fe2w, fpk, fp2p, ppk)
    return out if with_pos else (out[0], None)


def _node1_body(h_ref, h0_ref, p16_ref,
                pha_ref, phb_ref, phc_ref, phd_ref,
                ppa_ref, ppb_ref, ppc_ref, ppd_ref,
                fh1a_ref, fh1b_ref, fh2_ref, hpk_ref,
                fhp1a_ref, fhp1b_ref, fhp2_ref, ppk_ref,
                wa_ref, wb_ref, fpa_ref, fpb_ref, shift_ref,
                h1_ref, p1_ref, tsp_ref, tdp_ref, tse_ref, tde_ref):
    mh = (pha_ref[0] + phb_ref[0]) + (phc_ref[0] + phd_ref[0])
    mp = ((ppa_ref[0][:, 0:16] + ppb_ref[0][:, 0:16])
          + (ppc_ref[0][:, 0:16] + ppd_ref[0][:, 0:16])) * (1.0 / N)
    h = h_ref[...]
    u = _silu(jnp.dot(h, fh1a_ref[...], preferred_element_type=_f32)
              + jnp.dot(mh, fh1b_ref[...], preferred_element_type=_f32)
              + hpk_ref[0:1, :])
    hn = h + jnp.dot(u, fh2_ref[...], preferred_element_type=_f32) + hpk_ref[1:2, :]
    h1 = hn + h0_ref[...]
    p = p16_ref[...]
    tu = _silu(jnp.dot(p, fhp1a_ref[...], preferred_element_type=_f32)
               + jnp.dot(mp, fhp1b_ref[...], preferred_element_type=_f32)
               + ppk_ref[0:1, :])
    pn = p + jnp.dot(tu, fhp2_ref[...], preferred_element_type=_f32) + ppk_ref[1:2, :]
    psh = jnp.dot(pn, shift_ref[...], preferred_element_type=_f32)
    h1_ref[...] = h1
    p1_ref[...] = pn
    tsp_ref[...] = jnp.dot(h1, wa_ref[...], preferred_element_type=_f32)
    tdp_ref[...] = jnp.dot(h1, wb_ref[...], preferred_element_type=_f32)
    tse_ref[...] = jnp.dot(h1, fpa_ref[...], preferred_element_type=_f32) + psh
    tde_ref[...] = jnp.dot(h1, fpb_ref[...], preferred_element_type=_f32) + psh


def _tc_node1(h, h0, p16, pha, phb, ppa, ppb, fh1a, fh1b, fh2w, hpk,
              fhp1a, fhp1b, fhp2p, ppk, wa, wb, fpa, fpb, shiftm):
    g = NP // RB
    full = lambda r, c: pl.BlockSpec((r, c), lambda i: (0, 0))
    return pl.pallas_call(
        _node1_body,
        grid=(g,),
        in_specs=[
            pl.BlockSpec((RB, 128), lambda i: (i, 0)),
            pl.BlockSpec((RB, 128), lambda i: (i, 0)),
            pl.BlockSpec((RB, 16), lambda i: (i, 0)),
            pl.BlockSpec((1, RB, 128), lambda i: (0, i, 0)),
            pl.BlockSpec((1, RB, 128), lambda i: (1, i, 0)),
            pl.BlockSpec((1, RB, 128), lambda i: (0, i, 0)),
            pl.BlockSpec((1, RB, 128), lambda i: (1, i, 0)),
            pl.BlockSpec((1, RB, 128), lambda i: (0, i, 0)),
            pl.BlockSpec((1, RB, 128), lambda i: (1, i, 0)),
            pl.BlockSpec((1, RB, 128), lambda i: (0, i, 0)),
            pl.BlockSpec((1, RB, 128), lambda i: (1, i, 0)),
            full(128, 128), full(128, 128), full(128, 128), full(8, 128),
            full(16, 16), full(16, 16), full(16, 16), full(8, 16),
            full(128, 128), full(128, 128), full(128, 16), full(128, 16),
            full(16, 16),
        ],
        out_specs=[
            pl.BlockSpec((RB, 128), lambda i: (i, 0)),
            pl.BlockSpec((RB, 16), lambda i: (i, 0)),
            pl.BlockSpec((RB, 128), lambda i: (i, 0)),
            pl.BlockSpec((RB, 128), lambda i: (i, 0)),
            pl.BlockSpec((RB, 16), lambda i: (i, 0)),
            pl.BlockSpec((RB, 16), lambda i: (i, 0)),
        ],
        out_shape=[
            jax.ShapeDtypeStruct((NP, 128), _f32),
            jax.ShapeDtypeStruct((NP, 16), _f32),
            jax.ShapeDtypeStruct((NP, 128), _f32),
            jax.ShapeDtypeStruct((NP, 128), _f32),
            jax.ShapeDtypeStruct((NP, 16), _f32),
            jax.ShapeDtypeStruct((NP, 16), _f32),
        ],
    )(h, h0, p16, pha, pha, phb, phb, ppa, ppa, ppb, ppb,
      fh1a, fh1b, fh2w, hpk,
      fhp1a, fhp1b, fhp2p, ppk, wa, wb, fpa, fpb, shiftm)


def _node2_body(h_ref, h0_ref, pha_ref, phb_ref, phc_ref, phd_ref,
                fh1a_ref, fh1b_ref, fh2_ref, hpk_ref, out_ref):
    i = pl.program_id(0)
    mh = (pha_ref[0] + phb_ref[0]) + (phc_ref[0] + phd_ref[0])
    h = h_ref[...]
    u = _silu(jnp.dot(h, fh1a_ref[...], preferred_element_type=_f32)
              + jnp.dot(mh, fh1b_ref[...], preferred_element_type=_f32)
              + hpk_ref[0:1, :])
    hn = h + jnp.dot(u, fh2_ref[...], preferred_element_type=_f32) + hpk_ref[1:2, :]
    h2 = hn + h0_ref[...]
    rows = i * RB + lax.broadcasted_iota(jnp.int32, (RB, 1), 0)
    h2m = jnp.where(rows < N, h2, 0.0)
    s = jnp.sum(h2m, axis=0, keepdims=True) * (1.0 / N)

    @pl.when(i == 0)
    def _():
        out_ref[...] = s

    @pl.when(i > 0)
    def _():
        out_ref[...] += s


def _tc_node2(h, h0, pha, phb, fh1a, fh1b, fh2w, hpk):
    g = NP // RB
    full = lambda r, c: pl.BlockSpec((r, c), lambda i: (0, 0))
    return pl.pallas_call(
        _node2_body,
        grid=(g,),
        in_specs=[
            pl.BlockSpec((RB, 128), lambda i: (i, 0)),
            pl.BlockSpec((RB, 128), lambda i: (i, 0)),
            pl.BlockSpec((1, RB, 128), lambda i: (0, i, 0)),
            pl.BlockSpec((1, RB, 128), lambda i: (1, i, 0)),
            pl.BlockSpec((1, RB, 128), lambda i: (0, i, 0)),
            pl.BlockSpec((1, RB, 128), lambda i: (1, i, 0)),
            full(128, 128), full(128, 128), full(128, 128), full(8, 128),
        ],
        out_specs=pl.BlockSpec((1, 128), lambda i: (0, 0)),
        out_shape=jax.ShapeDtypeStruct((1, 128), _f32),
    )(h, h0, pha, pha, phb, phb, fh1a, fh1b, fh2w, hpk)


@functools.cache
def _make_sc_gather():
    mesh = plsc.VectorSubcoreMesh(core_axis_name="c", subcore_axis_name="s",
                                  num_cores=NC, num_subcores=NS)

    @functools.partial(
        pl.kernel,
        out_type=[jax.ShapeDtypeStruct((SLAB, 128), _f32),
                  jax.ShapeDtypeStruct((SLAB, 128), _f32),
                  jax.ShapeDtypeStruct((SLAB, 128), _f32)],
        mesh=mesh,
        scratch_types=[
            pltpu.VMEM((NCHS, CH), jnp.int32),
            pltpu.VMEM((NCHS, CH), jnp.int32),
            pltpu.VMEM((CH, 128), _f32),
            pltpu.VMEM((CH, 128), _f32),
            pltpu.VMEM((CH, 16), _f32),
            pltpu.VMEM((CH, 16), _f32),
            pltpu.VMEM((CH, 128), _f32),
            pltpu.VMEM((CH, 128), _f32),
            pltpu.VMEM((CH, 16), _f32),
            pltpu.VMEM((CH, 16), _f32),
            pltpu.SemaphoreType.DMA,
            pltpu.SemaphoreType.DMA,
        ],
        compiler_params=pltpu.CompilerParams(use_tc_tiling_on_sc=False),
    )
    def sc_gather(tsp_hbm, tdp_hbm, tse_hbm, tde_hbm, src_hbm, dst_hbm,
                  gsp_hbm, gdp_hbm, exc_hbm,
                  idxs_v, idxd_v,
                  b0sp, b0dp, b0se, b0de, b1sp, b1dp, b1se, b1de,
                  sem0, sem1):
        cid = lax.axis_index("c")
        sid = lax.axis_index("s")
        wid = sid * NC + cid
        pltpu.sync_copy(src_hbm.at[wid], idxs_v)
        pltpu.sync_copy(dst_hbm.at[wid], idxd_v)
        base = wid * EPWS
        tables = (tsp_hbm, tdp_hbm, tse_hbm, tde_hbm)
        idxes = (idxs_v, idxd_v, idxs_v, idxd_v)
        bufs = ((b0sp, b0dp, b0se, b0de), (b1sp, b1dp, b1se, b1de))

        def issue(j, p, sem):
            for t, b, idx in zip(tables, bufs[p], idxes):
                pltpu.async_copy(t.at[idx.at[j]], b, sem)

        def drain_write(j, p, sem):
            for t, b, idx in zip(tables, bufs[p], idxes):
                pltpu.make_async_copy(t.at[idx.at[j]], b, sem).wait()
            r = pl.ds(base + j * CH, CH)
            pltpu.sync_copy(bufs[p][0], gsp_hbm.at[r])
            pltpu.sync_copy(bufs[p][1], gdp_hbm.at[r])
            pltpu.sync_copy(bufs[p][2], exc_hbm.at[r, pl.ds(0, 16)])
            pltpu.sync_copy(bufs[p][3], exc_hbm.at[r, pl.ds(16, 16)])

        issue(0, 0, sem0)

        def body(k, carry):
            g0 = 2 * k
            issue(g0 + 1, 1, sem1)
            drain_write(g0, 0, sem0)
            issue(g0 + 2, 0, sem0)
            drain_write(g0 + 1, 1, sem1)
            return carry

        lax.fori_loop(0, (NCHS - 1) // 2, body, 0)
        drain_write(NCHS - 1, 0, sem0)

    return sc_gather


def _sc_gather(tsp, tdp, tse, tde, src2, dst2):
    return _make_sc_gather()(tsp, tdp, tse, tde, src2, dst2)


@functools.cache
def _make_sc_scatter(with_pos, slabs):
    nsl = len(slabs)
    mesh = plsc.VectorSubcoreMesh(core_axis_name="c", subcore_axis_name="s",
                                  num_cores=NC, num_subcores=NS)
    n_out = 2 if with_pos else 1
    scratch = [pltpu.VMEM_SHARED((N, 128), _f32)]
    if with_pos:
        scratch.append(pltpu.VMEM_SHARED((N, 16), _f32))
    scratch.append(pltpu.VMEM((NCHS, CH), jnp.int32))
    scratch += [pltpu.VMEM((CH, 128), _f32)] * 2
    if with_pos:
        scratch += [pltpu.VMEM((CH, 16), _f32)] * 2
    scratch += [pltpu.SemaphoreType.DMA] * 2

    @functools.partial(
        pl.kernel,
        out_type=[jax.ShapeDtypeStruct((NC, NP, 128), _f32)] * n_out,
        mesh=mesh,
        scratch_types=scratch,
        compiler_params=pltpu.CompilerParams(use_tc_tiling_on_sc=False),
    )
    def sc_scatter(*refs):
        it = iter(refs)
        chs = [next(it) for _ in range(nsl)]
        cps = [next(it) for _ in range(nsl)] if with_pos else [None] * nsl
        src_hbm = next(it)
        zeros_hbm = next(it)
        parth_hbm = next(it)
        partp_hbm = next(it) if with_pos else None
        acch_sh = next(it)
        accp_sh = next(it) if with_pos else None
        idx_v = next(it)
        bh0 = next(it)
        bh1 = next(it)
        bp0 = next(it) if with_pos else None
        bp1 = next(it) if with_pos else None
        s0 = next(it)
        s1 = next(it)
        cid = lax.axis_index("c")
        sid = lax.axis_index("s")
        wid = sid * NC + cid

        for k in range((NZC + NS - 1) // NS):
            c = sid + k * NS

            @pl.when(c < NZC)
            def _(c=c):
                r = pl.ds(c * RZ, RZ)
                pltpu.sync_copy(zeros_hbm.at[r], bh0)
                pltpu.sync_copy(bh0, acch_sh.at[r])
                if with_pos:
                    pltpu.sync_copy(zeros_hbm.at[r, pl.ds(0, 16)], bp0)
                    pltpu.sync_copy(bp0, accp_sh.at[r])

        plsc.subcore_barrier()
        base = wid * EPWS

        for i_sl, slab in enumerate(slabs):
            conh_hbm = chs[i_sl]
            conp_hbm = cps[i_sl]
            pltpu.sync_copy(src_hbm.at[slab, wid], idx_v)

            def issue(j, bh, bp, sem):
                r = pl.ds(base + j * CH, CH)
                pltpu.async_copy(conh_hbm.at[r], bh, sem)
                if with_pos:
                    pltpu.async_copy(conp_hbm.at[r, pl.ds(0, 16)], bp, sem)

            def drain_scatter(j, bh, bp, sem):
                r = pl.ds(base + j * CH, CH)
                pltpu.make_async_copy(conh_hbm.at[r], bh, sem).wait()
                if with_pos:
                    pltpu.make_async_copy(
                        conp_hbm.at[r, pl.ds(0, 16)], bp, sem).wait()
                pltpu.sync_copy(bh, acch_sh.at[idx_v.at[j]], add=True)
                if with_pos:
                    pltpu.sync_copy(bp, accp_sh.at[idx_v.at[j]], add=True)

            issue(0, bh0, bp0, s0)

            def body(k, carry):
                g0 = 2 * k
                issue(g0 + 1, bh1, bp1, s1)
                drain_scatter(g0, bh0, bp0, s0)
                issue(g0 + 2, bh0, bp0, s0)
                drain_scatter(g0 + 1, bh1, bp1, s1)
                return carry

            lax.fori_loop(0, (NCHS - 1) // 2, body, 0)
            drain_scatter(NCHS - 1, bh0, bp0, s0)
        plsc.subcore_barrier()

        for k in range((NZC + NS - 1) // NS):
            c = sid + k * NS

            @pl.when(c < NZC)
            def _(c=c):
                r = pl.ds(c * RZ, RZ)
                pltpu.sync_copy(acch_sh.at[r], bh0)
                pltpu.sync_copy(bh0, parth_hbm.at[cid, r])
                if with_pos:
                    pltpu.sync_copy(accp_sh.at[r], bp0)
                    pltpu.sync_copy(bp0, partp_hbm.at[cid, r, pl.ds(0, 16)])

    return sc_scatter


def _sc_scatter(chs, cps, src3, zeros128, slabs, with_pos=True):
    if with_pos:
        return _make_sc_scatter(True, slabs)(*chs, *cps, src3, zeros128)
    res = _make_sc_scatter(False, slabs)(*chs, src3, zeros128)
    return res[0] if isinstance(res, (list, tuple)) else res


def _conv_consts(p):
    fpk = jnp.zeros((8, 128), _f32)
    fpk = fpk.at[0].set(p["fe1_w"][256])
    fpk = fpk.at[1].set(p["fe1_b"])
    fpk = fpk.at[2].set(p["fe2_b"])
    fpk = fpk.at[3].set(p["inf_w"][:, 0])
    ppk = jnp.zeros((8, 16), _f32)
    ppk = ppk.at[0, 0:3].set(p["fp1_w"][256])
    ppk = ppk.at[1, 0:3].set(p["fp1_b"])
    ppk = ppk.at[2, 0:3].set(p["fp2_b"])
    ppk = ppk.at[3, 0:3].set(1.0)
    ppk = ppk.at[4, 3:6].set(1.0)
    ppk = ppk.at[5, 0].set(p["inf_b"][0])
    fp2p = jnp.zeros((16, 16), _f32).at[0:3, 0:3].set(p["fp2_w"])
    wa = p["fe1_w"][0:128]
    wb = p["fe1_w"][128:256]
    fpa = jnp.pad(p["fp1_w"][0:128], ((0, 0), (0, 13)))
    fpb = jnp.pad(p["fp1_w"][128:256], ((0, 0), (0, 13)))
    hpk = jnp.zeros((8, 128), _f32)
    hpk = hpk.at[0].set(p["fh1_b"])
    hpk = hpk.at[1].set(p["fh2_b"])
    fh1a = p["fh1_w"][0:128]
    fh1b = p["fh1_w"][128:256]
    fhp1a = jnp.zeros((16, 16), _f32).at[0:3, 0:3].set(p["fhp1_w"][0:3])
    fhp1b = jnp.zeros((16, 16), _f32).at[0:3, 0:3].set(p["fhp1_w"][3:6])
    fhp2p = jnp.zeros((16, 16), _f32).at[0:3, 0:3].set(p["fhp2_w"])
    ppku = jnp.zeros((8, 16), _f32)
    ppku = ppku.at[0, 0:3].set(p["fhp1_b"])
    ppku = ppku.at[1, 0:3].set(p["fhp2_b"])
    return dict(fpk=fpk, ppk=ppk, fp2p=fp2p, wa=wa, wb=wb, fpa=fpa, fpb=fpb,
                hpk=hpk, fh1a=fh1a, fh1b=fh1b, fhp1a=fhp1a, fhp1b=fhp1b,
                fhp2p=fhp2p, ppku=ppku, fe2w=p["fe2_w"])


def kernel(x, edge_index, pos, emb1, emb2, l1_w, l1_b, params):
    xp = jnp.pad(x.astype(jnp.int32), ((0, NP - N), (0, 0)))
    pos = pos.astype(_f32)
    possh = jnp.pad(pos, ((0, NP - N), (3, 10)))
    p16 = jnp.pad(pos, ((0, NP - N), (0, 13)))
    src3 = edge_index[:, 0].astype(jnp.int32).reshape(K, NW, NCHS, CH)
    dst3 = edge_index[:, 1].astype(jnp.int32).reshape(K, NW, NCHS, CH)
    zeros128 = jnp.zeros((N, 128), _f32)
    e1p = jnp.pad(emb1, ((0, 128 - emb1.shape[0]), (0, 0)))
    e2p = jnp.pad(emb2, ((0, 8 - emb2.shape[0]), (0, 0)))
    l1b2 = l1_b[None, :]
    shiftm = jnp.zeros((16, 16), _f32).at[0, 3].set(1.0).at[1, 4].set(1.0).at[2, 5].set(1.0)

    c1 = _conv_consts(params["c1"])
    c2 = _conv_consts(params["c2"])

    SL_A = (0, 1, 2)
    SL_B = (3, 4)

    def conv_edges(tsp, tdp, tse, tde, cc, with_pos=True):
        chs, cps = [], []
        for k in range(K):
            gsp, gdp, exc = _sc_gather(tsp, tdp, tse, tde, src3[k], dst3[k])
            ch, cp = _tc_edge(gsp, gdp, exc,
                              cc["fe2w"], cc["fpk"], cc["fp2p"], cc["ppk"],
                              with_pos=with_pos)
            chs.append(ch)
            cps.append(cp)
        pa = _sc_scatter(chs[0:3], cps[0:3], src3, zeros128, SL_A,
                         with_pos=with_pos)
        pb = _sc_scatter(chs[3:5], cps[3:5], src3, zeros128, SL_B,
                         with_pos=with_pos)
        return pa, pb

    h0, t1sp, t1dp, t1se, t1de = _tc_init(xp, possh, e1p, e2p, l1_w, l1b2,
                                          c1["wa"], c1["wb"], c1["fpa"], c1["fpb"])
    (ph1a, pp1a), (ph1b, pp1b) = conv_edges(t1sp, t1dp, t1se, t1de, c1)
    h1, p16_1, t2sp, t2dp, t2se, t2de = _tc_node1(
        h0, h0, p16, ph1a, ph1b, pp1a, pp1b,
        c1["fh1a"], c1["fh1b"], params["c1"]["fh2_w"], c1["hpk"],
        c1["fhp1a"], c1["fhp1b"], c1["fhp2p"], c1["ppku"],
        c2["wa"], c2["wb"], c2["fpa"], c2["fpb"], shiftm)
    ph2a, ph2b = conv_edges(t2sp, t2dp, t2se, t2de, c2, with_pos=False)
    out = _tc_node2(h1, h0, ph2a, ph2b,
                    c2["fh1a"], c2["fh1b"], params["c2"]["fh2_w"], c2["hpk"])
    return out


# speedup vs baseline: 3.0316x; 1.0046x over previous
import functools

import jax
import jax.numpy as jnp
from jax import lax
from jax.experimental import pallas as pl
from jax.experimental.pallas import tpu as pltpu
from jax.experimental.pallas import tpu_sc as plsc

N = 10000
NP = 10240
E = 320000
H = 128
RB = 1024
BE = 8000
NC = 2
NS = 16
NW = NC * NS
EPW = E // NW
CH = 80
NCH = EPW // CH
K = 5
SLAB = E // K
EPWS = SLAB // NW
NCHS = EPWS // CH
RZ = 80
NZC = N // RZ

_f32 = jnp.float32


def _silu(v):
    return v * jax.nn.sigmoid(v)


def _init_body(x_ref, possh_ref, e1_ref, e2_ref, l1w_ref, l1b_ref,
               wa_ref, wb_ref, fpa_ref, fpb_ref,
               h0_ref, tsp_ref, tdp_ref, tse_ref, tde_ref):
    xb = x_ref[...]
    oh1 = (xb[:, 0:1] == lax.broadcasted_iota(jnp.int32, (1, 128), 1)
           ).astype(_f32)
    oh2 = (xb[:, 1:2] == lax.broadcasted_iota(jnp.int32, (1, 8), 1)
           ).astype(_f32)
    hemb = (jnp.dot(oh1, e1_ref[...], preferred_element_type=_f32)
            + jnp.dot(oh2, e2_ref[...], preferred_element_type=_f32))
    h0 = jax.nn.relu(jnp.dot(hemb, l1w_ref[...], preferred_element_type=_f32)
                     + l1b_ref[...])
    possh = possh_ref[...]
    h0_ref[...] = h0
    tsp_ref[...] = jnp.dot(h0, wa_ref[...], preferred_element_type=_f32)
    tdp_ref[...] = jnp.dot(h0, wb_ref[...], preferred_element_type=_f32)
    tse_ref[...] = jnp.dot(h0, fpa_ref[...], preferred_element_type=_f32) + possh
    tde_ref[...] = jnp.dot(h0, fpb_ref[...], preferred_element_type=_f32) + possh


def _tc_init(xp, possh, e1p, e2p, l1w, l1b2, wa, wb, fpa, fpb):
    g = NP // RB
    return pl.pallas_call(
        _init_body,
        grid=(g,),
        in_specs=[
            pl.BlockSpec((RB, 2), lambda i: (i, 0)),
            pl.BlockSpec((RB, 16), lambda i: (i, 0)),
            pl.BlockSpec((128, 128), lambda i: (0, 0)),
            pl.BlockSpec((8, 128), lambda i: (0, 0)),
            pl.BlockSpec((128, 128), lambda i: (0, 0)),
            pl.BlockSpec((1, 128), lambda i: (0, 0)),
            pl.BlockSpec((128, 128), lambda i: (0, 0)),
            pl.BlockSpec((128, 128), lambda i: (0, 0)),
            pl.BlockSpec((128, 16), lambda i: (0, 0)),
            pl.BlockSpec((128, 16), lambda i: (0, 0)),
        ],
        out_specs=[
            pl.BlockSpec((RB, 128), lambda i: (i, 0)),
            pl.BlockSpec((RB, 128), lambda i: (i, 0)),
            pl.BlockSpec((RB, 128), lambda i: (i, 0)),
            pl.BlockSpec((RB, 16), lambda i: (i, 0)),
            pl.BlockSpec((RB, 16), lambda i: (i, 0)),
        ],
        out_shape=[
            jax.ShapeDtypeStruct((NP, 128), _f32),
            jax.ShapeDtypeStruct((NP, 128), _f32),
            jax.ShapeDtypeStruct((NP, 128), _f32),
            jax.ShapeDtypeStruct((NP, 16), _f32),
            jax.ShapeDtypeStruct((NP, 16), _f32),
        ],
    )(xp, possh, e1p, e2p, l1w, l1b2, wa, wb, fpa, fpb)


def _edge_body(with_pos, gsp_ref, gdp_ref, exc_ref,
               fe2w_ref, fpk_ref, fp2_ref, ppk_ref, outh_ref, outp_ref=None):
    a = gsp_ref[...] + gdp_ref[...]
    ex = exc_ref[...]
    es = ex[:, 0:16]
    ed = ex[:, 16:32]
    d16 = es - ed
    dd = d16 * ppk_ref[4:5, :]
    dist = jnp.sqrt(jnp.sum(dd * dd, axis=1, keepdims=True) + 1e-12)
    s1 = _silu(a + dist * fpk_ref[0:1, :] + fpk_ref[1:2, :])
    m = _silu(jnp.dot(s1, fe2w_ref[...], preferred_element_type=_f32)
              + fpk_ref[2:3, :])
    e = jax.nn.sigmoid(jnp.sum(m * fpk_ref[3:4, :], axis=1, keepdims=True)
                       + ppk_ref[5:6, 0:1])
    outh_ref[...] = e * m
    if with_pos:
        s16 = es + ed
        tp = _silu(s16 * ppk_ref[3:4, :] + dist * ppk_ref[0:1, :]
                   + ppk_ref[1:2, :])
        mp = _silu(jnp.dot(tp, fp2_ref[...], preferred_element_type=_f32)
                   + ppk_ref[2:3, :])
        outp_ref[...] = jnp.concatenate(
            [dist * mp, jnp.zeros((mp.shape[0], 112), _f32)], axis=1)


def _tc_edge(gsp, gdp, exc, fe2w, fpk, fp2p, ppk, with_pos=True):
    g = SLAB // BE
    n_out = 2 if with_pos else 1
    out = pl.pallas_call(
        functools.partial(_edge_body, with_pos),
        grid=(g,),
        in_specs=[
            pl.BlockSpec((BE, 128), lambda i: (i, 0)),
            pl.BlockSpec((BE, 128), lambda i: (i, 0)),
            pl.BlockSpec((BE, 128), lambda i: (i, 0)),
            pl.BlockSpec((128, 128), lambda i: (0, 0)),
            pl.BlockSpec((8, 128), lambda i: (0, 0)),
            pl.BlockSpec((16, 16), lambda i: (0, 0)),
            pl.BlockSpec((8, 16), lambda i: (0, 0)),
        ],
        out_specs=[pl.BlockSpec((BE, 128), lambda i: (i, 0))] * n_out,
        out_shape=[jax.ShapeDtypeStruct((SLAB, 128), _f32)] * n_out,
    )(gsp, gdp, exc, fe2w, fpk, fp2p, ppk)
    return out if with_pos else (out[0], None)


def _node1_body(h_ref, h0_ref, p16_ref,
                pha_ref, phb_ref, phc_ref, phd_ref,
                ppa_ref, ppb_ref, ppc_ref, ppd_ref,
                fh1a_ref, fh1b_ref, fh2_ref, hpk_ref,
                fhp1a_ref, fhp1b_ref, fhp2_ref, ppk_ref,
                wa_ref, wb_ref, fpa_ref, fpb_ref, shift_ref,
                h1_ref, p1_ref, tsp_ref, tdp_ref, tse_ref, tde_ref):
    mh = (pha_ref[0] + phb_ref[0]) + (phc_ref[0] + phd_ref[0])
    mp = ((ppa_ref[0][:, 0:16] + ppb_ref[0][:, 0:16])
          + (ppc_ref[0][:, 0:16] + ppd_ref[0][:, 0:16])) * (1.0 / N)
    h = h_ref[...]
    u = _silu(jnp.dot(h, fh1a_ref[...], preferred_element_type=_f32)
              + jnp.dot(mh, fh1b_ref[...], preferred_element_type=_f32)
              + hpk_ref[0:1, :])
    hn = h + jnp.dot(u, fh2_ref[...], preferred_element_type=_f32) + hpk_ref[1:2, :]
    h1 = hn + h0_ref[...]
    p = p16_ref[...]
    tu = _silu(jnp.dot(p, fhp1a_ref[...], preferred_element_type=_f32)
               + jnp.dot(mp, fhp1b_ref[...], preferred_element_type=_f32)
               + ppk_ref[0:1, :])
    pn = p + jnp.dot(tu, fhp2_ref[...], preferred_element_type=_f32) + ppk_ref[1:2, :]
    psh = jnp.dot(pn, shift_ref[...], preferred_element_type=_f32)
    h1_ref[...] = h1
    p1_ref[...] = pn
    tsp_ref[...] = jnp.dot(h1, wa_ref[...], preferred_element_type=_f32)
    tdp_ref[...] = jnp.dot(h1, wb_ref[...], preferred_element_type=_f32)
    tse_ref[...] = jnp.dot(h1, fpa_ref[...], preferred_element_type=_f32) + psh
    tde_ref[...] = jnp.dot(h1, fpb_ref[...], preferred_element_type=_f32) + psh


def _tc_node1(h, h0, p16, pha, phb, ppa, ppb, fh1a, fh1b, fh2w, hpk,
              fhp1a, fhp1b, fhp2p, ppk, wa, wb, fpa, fpb, shiftm):
    g = NP // RB
    full = lambda r, c: pl.BlockSpec((r, c), lambda i: (0, 0))
    return pl.pallas_call(
        _node1_body,
        grid=(g,),
        in_specs=[
            pl.BlockSpec((RB, 128), lambda i: (i, 0)),
            pl.BlockSpec((RB, 128), lambda i: (i, 0)),
            pl.BlockSpec((RB, 16), lambda i: (i, 0)),
            pl.BlockSpec((1, RB, 128), lambda i: (0, i, 0)),
            pl.BlockSpec((1, RB, 128), lambda i: (1, i, 0)),
            pl.BlockSpec((1, RB, 128), lambda i: (0, i, 0)),
            pl.BlockSpec((1, RB, 128), lambda i: (1, i, 0)),
            pl.BlockSpec((1, RB, 128), lambda i: (0, i, 0)),
            pl.BlockSpec((1, RB, 128), lambda i: (1, i, 0)),
            pl.BlockSpec((1, RB, 128), lambda i: (0, i, 0)),
            pl.BlockSpec((1, RB, 128), lambda i: (1, i, 0)),
            full(128, 128), full(128, 128), full(128, 128), full(8, 128),
            full(16, 16), full(16, 16), full(16, 16), full(8, 16),
            full(128, 128), full(128, 128), full(128, 16), full(128, 16),
            full(16, 16),
        ],
        out_specs=[
            pl.BlockSpec((RB, 128), lambda i: (i, 0)),
            pl.BlockSpec((RB, 16), lambda i: (i, 0)),
            pl.BlockSpec((RB, 128), lambda i: (i, 0)),
            pl.BlockSpec((RB, 128), lambda i: (i, 0)),
            pl.BlockSpec((RB, 16), lambda i: (i, 0)),
            pl.BlockSpec((RB, 16), lambda i: (i, 0)),
        ],
        out_shape=[
            jax.ShapeDtypeStruct((NP, 128), _f32),
            jax.ShapeDtypeStruct((NP, 16), _f32),
            jax.ShapeDtypeStruct((NP, 128), _f32),
            jax.ShapeDtypeStruct((NP, 128), _f32),
            jax.ShapeDtypeStruct((NP, 16), _f32),
            jax.ShapeDtypeStruct((NP, 16), _f32),
        ],
    )(h, h0, p16, pha, pha, phb, phb, ppa, ppa, ppb, ppb,
      fh1a, fh1b, fh2w, hpk,
      fhp1a, fhp1b, fhp2p, ppk, wa, wb, fpa, fpb, shiftm)


def _node2_body(h_ref, h0_ref, pha_ref, phb_ref, phc_ref, phd_ref,
                fh1a_ref, fh1b_ref, fh2_ref, hpk_ref, out_ref):
    i = pl.program_id(0)
    mh = (pha_ref[0] + phb_ref[0]) + (phc_ref[0] + phd_ref[0])
    h = h_ref[...]
    u = _silu(jnp.dot(h, fh1a_ref[...], preferred_element_type=_f32)
              + jnp.dot(mh, fh1b_ref[...], preferred_element_type=_f32)
              + hpk_ref[0:1, :])
    hn = h + jnp.dot(u, fh2_ref[...], preferred_element_type=_f32) + hpk_ref[1:2, :]
    h2 = hn + h0_ref[...]
    rows = i * RB + lax.broadcasted_iota(jnp.int32, (RB, 1), 0)
    h2m = jnp.where(rows < N, h2, 0.0)
    s = jnp.sum(h2m, axis=0, keepdims=True) * (1.0 / N)

    @pl.when(i == 0)
    def _():
        out_ref[...] = s

    @pl.when(i > 0)
    def _():
        out_ref[...] += s


def _tc_node2(h, h0, pha, phb, fh1a, fh1b, fh2w, hpk):
    g = NP // RB
    full = lambda r, c: pl.BlockSpec((r, c), lambda i: (0, 0))
    return pl.pallas_call(
        _node2_body,
        grid=(g,),
        in_specs=[
            pl.BlockSpec((RB, 128), lambda i: (i, 0)),
            pl.BlockSpec((RB, 128), lambda i: (i, 0)),
            pl.BlockSpec((1, RB, 128), lambda i: (0, i, 0)),
            pl.BlockSpec((1, RB, 128), lambda i: (1, i, 0)),
            pl.BlockSpec((1, RB, 128), lambda i: (0, i, 0)),
            pl.BlockSpec((1, RB, 128), lambda i: (1, i, 0)),
            full(128, 128), full(128, 128), full(128, 128), full(8, 128),
        ],
        out_specs=pl.BlockSpec((1, 128), lambda i: (0, 0)),
        out_shape=jax.ShapeDtypeStruct((1, 128), _f32),
    )(h, h0, pha, pha, phb, phb, fh1a, fh1b, fh2w, hpk)


@functools.cache
def _make_sc_gather():
    mesh = plsc.VectorSubcoreMesh(core_axis_name="c", subcore_axis_name="s",
                                  num_cores=NC, num_subcores=NS)

    @functools.partial(
        pl.kernel,
        out_type=[jax.ShapeDtypeStruct((SLAB, 128), _f32),
                  jax.ShapeDtypeStruct((SLAB, 128), _f32),
                  jax.ShapeDtypeStruct((SLAB, 128), _f32)],
        mesh=mesh,
        scratch_types=[
            pltpu.VMEM((NCHS, CH), jnp.int32),
            pltpu.VMEM((NCHS, CH), jnp.int32),
            pltpu.VMEM((CH, 128), _f32),
            pltpu.VMEM((CH, 128), _f32),
            pltpu.VMEM((CH, 16), _f32),
            pltpu.VMEM((CH, 16), _f32),
            pltpu.VMEM((CH, 128), _f32),
            pltpu.VMEM((CH, 128), _f32),
            pltpu.VMEM((CH, 16), _f32),
            pltpu.VMEM((CH, 16), _f32),
            pltpu.SemaphoreType.DMA,
            pltpu.SemaphoreType.DMA,
        ],
        compiler_params=pltpu.CompilerParams(use_tc_tiling_on_sc=False),
    )
    def sc_gather(tsp_hbm, tdp_hbm, tse_hbm, tde_hbm, src_hbm, dst_hbm,
                  gsp_hbm, gdp_hbm, exc_hbm,
                  idxs_v, idxd_v,
                  b0sp, b0dp, b0se, b0de, b1sp, b1dp, b1se, b1de,
                  sem0, sem1):
        cid = lax.axis_index("c")
        sid = lax.axis_index("s")
        wid = sid * NC + cid
        pltpu.sync_copy(src_hbm.at[wid], idxs_v)
        pltpu.sync_copy(dst_hbm.at[wid], idxd_v)
        base = wid * EPWS
        tables = (tsp_hbm, tdp_hbm, tse_hbm, tde_hbm)
        idxes = (idxs_v, idxd_v, idxs_v, idxd_v)
        bufs = ((b0sp, b0dp, b0se, b0de), (b1sp, b1dp, b1se, b1de))

        def issue(j, p, sem):
            for t, b, idx in zip(tables, bufs[p], idxes):
                pltpu.async_copy(t.at[idx.at[j]], b, sem)

        def drain_write(j, p, sem):
            for t, b, idx in zip(tables, bufs[p], idxes):
                pltpu.make_async_copy(t.at[idx.at[j]], b, sem).wait()
            r = pl.ds(base + j * CH, CH)
            pltpu.sync_copy(bufs[p][0], gsp_hbm.at[r])
            pltpu.sync_copy(bufs[p][1], gdp_hbm.at[r])
            pltpu.sync_copy(bufs[p][2], exc_hbm.at[r, pl.ds(0, 16)])
            pltpu.sync_copy(bufs[p][3], exc_hbm.at[r, pl.ds(16, 16)])

        issue(0, 0, sem0)

        def body(k, carry):
            g0 = 2 * k
            issue(g0 + 1, 1, sem1)
            drain_write(g0, 0, sem0)
            issue(g0 + 2, 0, sem0)
            drain_write(g0 + 1, 1, sem1)
            return carry

        lax.fori_loop(0, (NCHS - 1) // 2, body, 0)
        drain_write(NCHS - 1, 0, sem0)

    return sc_gather


def _sc_gather(tsp, tdp, tse, tde, src2, dst2):
    return _make_sc_gather()(tsp, tdp, tse, tde, src2, dst2)


@functools.cache
def _make_sc_scatter(with_pos, slabs):
    nsl = len(slabs)
    mesh = plsc.VectorSubcoreMesh(core_axis_name="c", subcore_axis_name="s",
                                  num_cores=NC, num_subcores=NS)
    n_out = 2 if with_pos else 1
    scratch = [pltpu.VMEM_SHARED((N, 128), _f32)]
    if with_pos:
        scratch.append(pltpu.VMEM_SHARED((N, 16), _f32))
    scratch.append(pltpu.VMEM((NCHS, CH), jnp.int32))
    scratch += [pltpu.VMEM((CH, 128), _f32)] * 2
    if with_pos:
        scratch += [pltpu.VMEM((CH, 16), _f32)] * 2
    scratch += [pltpu.SemaphoreType.DMA] * 2

    @functools.partial(
        pl.kernel,
        out_type=[jax.ShapeDtypeStruct((NC, NP, 128), _f32)] * n_out,
        mesh=mesh,
        scratch_types=scratch,
        compiler_params=pltpu.CompilerParams(use_tc_tiling_on_sc=False),
    )
    def sc_scatter(*refs):
        it = iter(refs)
        chs = [next(it) for _ in range(nsl)]
        cps = [next(it) for _ in range(nsl)] if with_pos else [None] * nsl
        src_hbm = next(it)
        zeros_hbm = next(it)
        parth_hbm = next(it)
        partp_hbm = next(it) if with_pos else None
        acch_sh = next(it)
        accp_sh = next(it) if with_pos else None
        idx_v = next(it)
        bh0 = next(it)
        bh1 = next(it)
        bp0 = next(it) if with_pos else None
        bp1 = next(it) if with_pos else None
        s0 = next(it)
        s1 = next(it)
        cid = lax.axis_index("c")
        sid = lax.axis_index("s")
        wid = sid * NC + cid

        for k in range((NZC + NS - 1) // NS):
            c = sid + k * NS

            @pl.when(c < NZC)
            def _(c=c):
                r = pl.ds(c * RZ, RZ)
                pltpu.sync_copy(zeros_hbm.at[r], bh0)
                pltpu.sync_copy(bh0, acch_sh.at[r])
                if with_pos:
                    pltpu.sync_copy(zeros_hbm.at[r, pl.ds(0, 16)], bp0)
                    pltpu.sync_copy(bp0, accp_sh.at[r])

        plsc.subcore_barrier()
        base = wid * EPWS

        for i_sl, slab in enumerate(slabs):
            conh_hbm = chs[i_sl]
            conp_hbm = cps[i_sl]
            pltpu.sync_copy(src_hbm.at[slab, wid], idx_v)

            def issue(j, bh, bp, sem):
                r = pl.ds(base + j * CH, CH)
                pltpu.async_copy(conh_hbm.at[r], bh, sem)
                if with_pos:
                    pltpu.async_copy(conp_hbm.at[r, pl.ds(0, 16)], bp, sem)

            def drain_scatter(j, bh, bp, sem):
                r = pl.ds(base + j * CH, CH)
                pltpu.make_async_copy(conh_hbm.at[r], bh, sem).wait()
                if with_pos:
                    pltpu.make_async_copy(
                        conp_hbm.at[r, pl.ds(0, 16)], bp, sem).wait()
                pltpu.sync_copy(bh, acch_sh.at[idx_v.at[j]], add=True)
                if with_pos:
                    pltpu.sync_copy(bp, accp_sh.at[idx_v.at[j]], add=True)

            issue(0, bh0, bp0, s0)

            def body(k, carry):
                g0 = 2 * k
                issue(g0 + 1, bh1, bp1, s1)
                drain_scatter(g0, bh0, bp0, s0)
                issue(g0 + 2, bh0, bp0, s0)
                drain_scatter(g0 + 1, bh1, bp1, s1)
                return carry

            lax.fori_loop(0, (NCHS - 1) // 2, body, 0)
            drain_scatter(NCHS - 1, bh0, bp0, s0)
        plsc.subcore_barrier()

        for k in range((NZC + NS - 1) // NS):
            c = sid + k * NS

            @pl.when(c < NZC)
            def _(c=c):
                r = pl.ds(c * RZ, RZ)
                pltpu.sync_copy(acch_sh.at[r], bh0)
                pltpu.sync_copy(bh0, parth_hbm.at[cid, r])
                if with_pos:
                    pltpu.sync_copy(accp_sh.at[r], bp0)
                    pltpu.sync_copy(bp0, partp_hbm.at[cid, r, pl.ds(0, 16)])

    return sc_scatter


def _sc_scatter(chs, cps, src3, zeros128, slabs, with_pos=True):
    if with_pos:
        return _make_sc_scatter(True, slabs)(*chs, *cps, src3, zeros128)
    res = _make_sc_scatter(False, slabs)(*chs, src3, zeros128)
    return res[0] if isinstance(res, (list, tuple)) else res


def _conv_consts(p):
    fpk = jnp.zeros((8, 128), _f32)
    fpk = fpk.at[0].set(p["fe1_w"][256])
    fpk = fpk.at[1].set(p["fe1_b"])
    fpk = fpk.at[2].set(p["fe2_b"])
    fpk = fpk.at[3].set(p["inf_w"][:, 0])
    ppk = jnp.zeros((8, 16), _f32)
    ppk = ppk.at[0, 0:3].set(p["fp1_w"][256])
    ppk = ppk.at[1, 0:3].set(p["fp1_b"])
    ppk = ppk.at[2, 0:3].set(p["fp2_b"])
    ppk = ppk.at[3, 0:3].set(1.0)
    ppk = ppk.at[4, 3:6].set(1.0)
    ppk = ppk.at[5, 0].set(p["inf_b"][0])
    fp2p = jnp.zeros((16, 16), _f32).at[0:3, 0:3].set(p["fp2_w"])
    wa = p["fe1_w"][0:128]
    wb = p["fe1_w"][128:256]
    fpa = jnp.pad(p["fp1_w"][0:128], ((0, 0), (0, 13)))
    fpb = jnp.pad(p["fp1_w"][128:256], ((0, 0), (0, 13)))
    hpk = jnp.zeros((8, 128), _f32)
    hpk = hpk.at[0].set(p["fh1_b"])
    hpk = hpk.at[1].set(p["fh2_b"])
    fh1a = p["fh1_w"][0:128]
    fh1b = p["fh1_w"][128:256]
    fhp1a = jnp.zeros((16, 16), _f32).at[0:3, 0:3].set(p["fhp1_w"][0:3])
    fhp1b = jnp.zeros((16, 16), _f32).at[0:3, 0:3].set(p["fhp1_w"][3:6])
    fhp2p = jnp.zeros((16, 16), _f32).at[0:3, 0:3].set(p["fhp2_w"])
    ppku = jnp.zeros((8, 16), _f32)
    ppku = ppku.at[0, 0:3].set(p["fhp1_b"])
    ppku = ppku.at[1, 0:3].set(p["fhp2_b"])
    return dict(fpk=fpk, ppk=ppk, fp2p=fp2p, wa=wa, wb=wb, fpa=fpa, fpb=fpb,
                hpk=hpk, fh1a=fh1a, fh1b=fh1b, fhp1a=fhp1a, fhp1b=fhp1b,
                fhp2p=fhp2p, ppku=ppku, fe2w=p["fe2_w"])


def kernel(x, edge_index, pos, emb1, emb2, l1_w, l1_b, params):
    xp = jnp.pad(x.astype(jnp.int32), ((0, NP - N), (0, 0)))
    pos = pos.astype(_f32)
    possh = jnp.pad(pos, ((0, NP - N), (3, 10)))
    p16 = jnp.pad(pos, ((0, NP - N), (0, 13)))
    src3 = edge_index[:, 0].astype(jnp.int32).reshape(K, NW, NCHS, CH)
    dst3 = edge_index[:, 1].astype(jnp.int32).reshape(K, NW, NCHS, CH)
    zeros128 = jnp.zeros((N, 128), _f32)
    e1p = jnp.pad(emb1, ((0, 128 - emb1.shape[0]), (0, 0)))
    e2p = jnp.pad(emb2, ((0, 8 - emb2.shape[0]), (0, 0)))
    l1b2 = l1_b[None, :]
    shiftm = jnp.zeros((16, 16), _f32).at[0, 3].set(1.0).at[1, 4].set(1.0).at[2, 5].set(1.0)

    c1 = _conv_consts(params["c1"])
    c2 = _conv_consts(params["c2"])

    SL_A = (0, 1, 2)
    SL_B = (3, 4)

    def conv_edges(tsp, tdp, tse, tde, cc, with_pos=True):
        chs, cps = [], []
        for k in range(K):
            gsp, gdp, exc = _sc_gather(tsp, tdp, tse, tde, src3[k], dst3[k])
            ch, cp = _tc_edge(gsp, gdp, exc,
                              cc["fe2w"], cc["fpk"], cc["fp2p"], cc["ppk"],
                              with_pos=with_pos)
            chs.append(ch)
            cps.append(cp)
        pa = _sc_scatter(chs[0:3], cps[0:3], src3, zeros128, SL_A,
                         with_pos=with_pos)
        pb = _sc_scatter(chs[3:5], cps[3:5], src3, zeros128, SL_B,
                         with_pos=with_pos)
        return pa, pb

    h0, t1sp, t1dp, t1se, t1de = _tc_init(xp, possh, e1p, e2p, l1_w, l1b2,
                                          c1["wa"], c1["wb"], c1["fpa"], c1["fpb"])
    (ph1a, pp1a), (ph1b, pp1b) = conv_edges(t1sp, t1dp, t1se, t1de, c1)
    h1, p16_1, t2sp, t2dp, t2se, t2de = _tc_node1(
        h0, h0, p16, ph1a, ph1b, pp1a, pp1b,
        c1["fh1a"], c1["fh1b"], params["c1"]["fh2_w"], c1["hpk"],
        c1["fhp1a"], c1["fhp1b"], c1["fhp2p"], c1["ppku"],
        c2["wa"], c2["wb"], c2["fpa"], c2["fpb"], shiftm)
    ph2a, ph2b = conv_edges(t2sp, t2dp, t2se, t2de, c2, with_pos=False)
    out = _tc_node2(h1, h0, ph2a, ph2b,
                    c2["fh1a"], c2["fh1b"], params["c2"]["fh2_w"], c2["hpk"])
    return out
